# baseline (speedup 1.0000x reference)
import functools

import jax
import jax.numpy as jnp
from jax import lax
from jax.experimental import pallas as pl
from jax.experimental.pallas import tpu as pltpu

_B = 256
_NEL = 19
_N = _B * _NEL
_D0 = 20000
_WLEN = 25
_NPOOL = 800
_CHUNK = 3200
_NCHUNK = 6
_R = 152
_GRID = _N // _R


def _pool_mats():
    j = jnp.arange(_CHUNK)
    pc = (j[:, None] // _WLEN == jnp.arange(128)[None, :]).astype(jnp.float32) / _WLEN
    jt = jnp.arange(_D0 - _NCHUNK * _CHUNK)
    pt = (jt[:, None] // _WLEN == jnp.arange(128)[None, :]).astype(jnp.float32) / _WLEN
    return pc, pt


def _k1_body(x_ref, pc_ref, pt_ref, w2_ref, b2_ref, h1_ref, s_ref, q_ref, pooled_ref):
    for c in range(_NCHUNK):
        pooled_ref[:, 128 * c:128 * (c + 1)] = jnp.dot(
            x_ref[:, _CHUNK * c:_CHUNK * (c + 1)], pc_ref[...],
            preferred_element_type=jnp.float32)
    pooled_ref[:, _NCHUNK * 128:(_NCHUNK + 1) * 128] = jnp.dot(
        x_ref[:, _NCHUNK * _CHUNK:_D0], pt_ref[...],
        preferred_element_type=jnp.float32)
    h = jnp.dot(pooled_ref[...], w2_ref[...], preferred_element_type=jnp.float32)
    h = jnp.maximum(h + b2_ref[...], 0.0)
    h1_ref[...] = h
    h3 = h.reshape(_R // _NEL, _NEL, 512)
    ps = jnp.sum(h3, axis=(0, 2)).reshape(1, _NEL)
    pq = jnp.sum(h3 * h3, axis=(0, 2)).reshape(1, _NEL)

    @pl.when(pl.program_id(0) == 0)
    def _init():
        s_ref[...] = jnp.zeros_like(s_ref)
        q_ref[...] = jnp.zeros_like(q_ref)

    s_ref[...] += ps
    q_ref[...] += pq


def _stage1(x, w2pad, b2):
    pc, pt = _pool_mats()
    return pl.pallas_call(
        _k1_body,
        grid=(_GRID,),
        in_specs=[
            pl.BlockSpec((_R, _D0), lambda i: (i, 0)),
            pl.BlockSpec((_CHUNK, 128), lambda i: (0, 0)),
            pl.BlockSpec((_D0 - _NCHUNK * _CHUNK, 128), lambda i: (0, 0)),
            pl.BlockSpec(((_NCHUNK + 1) * 128, 512), lambda i: (0, 0)),
            pl.BlockSpec((1, 512), lambda i: (0, 0)),
        ],
        out_specs=[
            pl.BlockSpec((_R, 512), lambda i: (i, 0)),
            pl.BlockSpec((1, _NEL), lambda i: (0, 0)),
            pl.BlockSpec((1, _NEL), lambda i: (0, 0)),
        ],
        out_shape=[
            jax.ShapeDtypeStruct((_N, 512), jnp.float32),
            jax.ShapeDtypeStruct((1, _NEL), jnp.float32),
            jax.ShapeDtypeStruct((1, _NEL), jnp.float32),
        ],
        scratch_shapes=[pltpu.VMEM((_R, (_NCHUNK + 1) * 128), jnp.float32)],
    )(x, pc, pt, w2pad, b2)


def _bn_scale_shift(s, q, count, g, be):
    m = s / count
    v = q / count - m * m
    inv = lax.rsqrt(v + 1e-5) * g
    return inv, be - m * inv


def _mk_mlp_bn(din, dout):
    def body(h_ref, sc_ref, sh_ref, w_ref, b_ref, out_ref, s_ref, q_ref):
        hb = h_ref[...] * sc_ref[...] + sh_ref[...]
        h2 = jnp.dot(hb, w_ref[...], preferred_element_type=jnp.float32)
        h2 = jnp.maximum(h2 + b_ref[...], 0.0)
        out_ref[...] = h2
        h3 = h2.reshape(_R // _NEL, _NEL, dout)
        ps = jnp.sum(h3, axis=(0, 2)).reshape(1, _NEL)
        pq = jnp.sum(h3 * h3, axis=(0, 2)).reshape(1, _NEL)

        @pl.when(pl.program_id(0) == 0)
        def _init():
            s_ref[...] = jnp.zeros_like(s_ref)
            q_ref[...] = jnp.zeros_like(q_ref)

        s_ref[...] += ps
        q_ref[...] += pq

    def call(h, sc_col, sh_col, w, b):
        return pl.pallas_call(
            body,
            grid=(_GRID,),
            in_specs=[
                pl.BlockSpec((_R, din), lambda i: (i, 0)),
                pl.BlockSpec((_R, 1), lambda i: (i, 0)),
                pl.BlockSpec((_R, 1), lambda i: (i, 0)),
                pl.BlockSpec((din, dout), lambda i: (0, 0)),
                pl.BlockSpec((1, dout), lambda i: (0, 0)),
            ],
            out_specs=[
                pl.BlockSpec((_R, dout), lambda i: (i, 0)),
                pl.BlockSpec((1, _NEL), lambda i: (0, 0)),
                pl.BlockSpec((1, _NEL), lambda i: (0, 0)),
            ],
            out_shape=[
                jax.ShapeDtypeStruct((_N, dout), jnp.float32),
                jax.ShapeDtypeStruct((1, _NEL), jnp.float32),
                jax.ShapeDtypeStruct((1, _NEL), jnp.float32),
            ],
        )(h, sc_col, sh_col, w, b)

    return call


def _mk_apply_root(din, dmid, col_scale):
    def body(h_ref, sc_ref, sh_ref, wr_ref, hn_ref, root_ref):
        hn = h_ref[...] * sc_ref[...] + sh_ref[...]
        hn_ref[...] = hn
        root_ref[...] = jnp.dot(hn, wr_ref[...], preferred_element_type=jnp.float32)

    sspec = (pl.BlockSpec((_R, 1), lambda i: (i, 0)) if col_scale
             else pl.BlockSpec((1, din), lambda i: (0, 0)))

    def call(h, sc, sh, wroot):
        return pl.pallas_call(
            body,
            grid=(_GRID,),
            in_specs=[
                pl.BlockSpec((_R, din), lambda i: (i, 0)),
                sspec, sspec,
                pl.BlockSpec((din, dmid), lambda i: (0, 0)),
            ],
            out_specs=[
                pl.BlockSpec((_R, din), lambda i: (i, 0)),
                pl.BlockSpec((_R, dmid), lambda i: (i, 0)),
            ],
            out_shape=[
                jax.ShapeDtypeStruct((_N, din), jnp.float32),
                jax.ShapeDtypeStruct((_N, dmid), jnp.float32),
            ],
        )(h, sc, sh, wroot)

    return call


def _mk_combine(din, dout):
    def body(a0_ref, a1_ref, root_ref, wrel_ref, brel_ref, out_ref, s_ref, q_ref):
        agg = a0_ref[...] + a1_ref[...]
        h = jnp.dot(agg, wrel_ref[...], preferred_element_type=jnp.float32)
        h = jnp.maximum(h + brel_ref[...] + root_ref[...], 0.0)
        out_ref[...] = h
        ps = jnp.sum(h, axis=0).reshape(1, dout)
        pq = jnp.sum(h * h, axis=0).reshape(1, dout)

        @pl.when(pl.program_id(0) == 0)
        def _init():
            s_ref[...] = jnp.zeros_like(s_ref)
            q_ref[...] = jnp.zeros_like(q_ref)

        s_ref[...] += ps
        q_ref[...] += pq

    def call(a0, a1, root, wrel, brel):
        return pl.pallas_call(
            body,
            grid=(_GRID,),
            in_specs=[
                pl.BlockSpec((_R, din), lambda i: (i, 0)),
                pl.BlockSpec((_R, din), lambda i: (i, 0)),
                pl.BlockSpec((_R, dout), lambda i: (i, 0)),
                pl.BlockSpec((din, dout), lambda i: (0, 0)),
                pl.BlockSpec((1, dout), lambda i: (0, 0)),
            ],
            out_specs=[
                pl.BlockSpec((_R, dout), lambda i: (i, 0)),
                pl.BlockSpec((1, dout), lambda i: (0, 0)),
                pl.BlockSpec((1, dout), lambda i: (0, 0)),
            ],
            out_shape=[
                jax.ShapeDtypeStruct((_N, dout), jnp.float32),
                jax.ShapeDtypeStruct((1, dout), jnp.float32),
                jax.ShapeDtypeStruct((1, dout), jnp.float32),
            ],
        )(a0, a1, root, wrel, brel)

    return call


def _pool_body(h_ref, sc_ref, sh_ref, out_ref):
    hn = h_ref[...] * sc_ref[...] + sh_ref[...]
    out_ref[...] = jnp.max(hn.reshape(_R // _NEL, _NEL, 64), axis=1)


def _maxpool(h, sc, sh):
    return pl.pallas_call(
        _pool_body,
        grid=(_GRID,),
        in_specs=[
            pl.BlockSpec((_R, 64), lambda i: (i, 0)),
            pl.BlockSpec((1, 64), lambda i: (0, 0)),
            pl.BlockSpec((1, 64), lambda i: (0, 0)),
        ],
        out_specs=pl.BlockSpec((_R // _NEL, 64), lambda i: (i, 0)),
        out_shape=jax.ShapeDtypeStruct((_B, 64), jnp.float32),
    )(h, sc, sh)


def _heads_body(p_ref, w5_ref, b5_ref, wc_ref, bc_ref, out_ref):
    feat = jnp.dot(p_ref[...], w5_ref[...], preferred_element_type=jnp.float32)
    feat = jnp.maximum(feat + b5_ref[...], 0.0)
    z = jnp.dot(feat, wc_ref[...], preferred_element_type=jnp.float32) + bc_ref[...]
    ps = []
    for k in range(3):
        zp = z[:, 2 * k:2 * k + 2]
        m = jnp.max(zp, axis=1, keepdims=True)
        e = jnp.exp(zp - m)
        ps.append(e / jnp.sum(e, axis=1, keepdims=True))
    p0, p1, p2 = ps
    p_hc = p0[:, 0:1] * p1[:, 0:1]
    p_ad = p0[:, 1:2] * p2[:, 1:2]
    p_ftd = p0[:, 0:1] * p1[:, 1:2] + p0[:, 1:2] * p2[:, 0:1]
    out_ref[...] = jnp.log(jnp.concatenate([p_hc, p_ftd, p_ad], axis=1) + 1e-8)


def _heads(pooled, w5, b5, wcat, bcat):
    return pl.pallas_call(
        _heads_body,
        in_specs=[pl.BlockSpec(pooled.shape, lambda: (0, 0))] * 1 + [
            pl.BlockSpec(w5.shape, lambda: (0, 0)),
            pl.BlockSpec(b5.shape, lambda: (0, 0)),
            pl.BlockSpec(wcat.shape, lambda: (0, 0)),
            pl.BlockSpec(bcat.shape, lambda: (0, 0)),
        ],
        out_specs=pl.BlockSpec((_B, 3), lambda: (0, 0)),
        out_shape=jax.ShapeDtypeStruct((_B, 3), jnp.float32),
    )(pooled, w5, b5, wcat, bcat)


def kernel(x, edge_index, batch, W2, b2, g3, be3, W3, b3, g4, be4, W4, b4,
           g5, be5, ew1, Wrel1, brel1, Wroot1, g6, be6, ew2, Wrel2, brel2,
           Wroot2, g7, be7, W5, b5, Whr, bhr, Whf, bhf, Wfa, bfa):
    w2pad = jnp.concatenate(
        [W2, jnp.zeros(((_NCHUNK + 1) * 128 - _NPOOL, 512), jnp.float32)], axis=0)
    h1, s1, q1 = _stage1(x, w2pad, b2.reshape(1, 512))

    def col(v19):
        return jnp.tile(v19, _B)[:, None]

    inv1, sh1 = _bn_scale_shift(s1[0], q1[0], _B * 512.0, g3, be3)
    h2, s2, q2 = _mk_mlp_bn(512, 256)(h1, col(inv1), col(sh1), W3,
                                      b3.reshape(1, 256))
    inv2, sh2 = _bn_scale_shift(s2[0], q2[0], _B * 256.0, g4, be4)
    h3, s3, q3 = _mk_mlp_bn(256, 128)(h2, col(inv2), col(sh2), W4,
                                      b4.reshape(1, 128))
    inv3, sh3 = _bn_scale_shift(s3[0], q3[0], _B * 128.0, g5, be5)
    h3n, root1 = _mk_apply_root(128, 64, True)(h3, col(inv3), col(sh3), Wroot1)

    src = edge_index[0]
    dst = edge_index[1]

    def gconv_agg(hh, ew):
        w_exp = jnp.tile(jax.nn.softplus(ew), _B)
        msg = hh[src] * w_exp[:, None]
        return jnp.zeros_like(hh).at[dst].add(msg)

    agg1 = gconv_agg(h3n, ew1)
    z1 = jnp.zeros_like(agg1)
    h4, s4, q4 = _mk_combine(128, 64)(agg1, z1, root1, Wrel1,
                                      brel1.reshape(1, 64))
    inv4, sh4 = _bn_scale_shift(s4[0:1], q4[0:1], float(_N), g6.reshape(1, 64),
                                be6.reshape(1, 64))
    h4n, root2 = _mk_apply_root(64, 64, False)(h4, inv4, sh4, Wroot2)

    agg2 = gconv_agg(h4n, ew2)
    z2 = jnp.zeros_like(agg2)
    h5, s5, q5 = _mk_combine(64, 64)(agg2, z2, root2, Wrel2,
                                     brel2.reshape(1, 64))
    inv5, sh5 = _bn_scale_shift(s5[0:1], q5[0:1], float(_N), g7.reshape(1, 64),
                                be7.reshape(1, 64))
    pooled = _maxpool(h5, inv5, sh5)
    wcat = jnp.concatenate([Whr, Whf, Wfa], axis=1)
    bcat = jnp.concatenate([bhr, bhf, bfa]).reshape(1, 6)
    return _heads(pooled, W5, b5.reshape(1, 32), wcat, bcat)


# speedup vs baseline: 1.2928x; 1.2928x over previous
import functools

import jax
import jax.numpy as jnp
from jax import lax
from jax.experimental import pallas as pl
from jax.experimental.pallas import tpu as pltpu
from jax.experimental.pallas import tpu_sc as plsc

_B = 256
_NEL = 19
_N = _B * _NEL
_D0 = 20000
_WLEN = 25
_NPOOL = 800
_CHUNK = 3200
_NCHUNK = 6
_R = 152
_GRID = _N // _R


def _pool_mats():
    j = jnp.arange(_CHUNK)
    pc = (j[:, None] // _WLEN == jnp.arange(128)[None, :]).astype(jnp.float32) / _WLEN
    jt = jnp.arange(_D0 - _NCHUNK * _CHUNK)
    pt = (jt[:, None] // _WLEN == jnp.arange(128)[None, :]).astype(jnp.float32) / _WLEN
    return pc, pt


def _k1_body(x_ref, pc_ref, pt_ref, w2_ref, b2_ref, ew1_ref, ew2_ref,
             h1_ref, s_ref, q_ref, w1t_ref, w2t_ref, pooled_ref):
    @pl.when(pl.program_id(0) == 0)
    def _wexp():
        for ew_ref, wt_ref in ((ew1_ref, w1t_ref), (ew2_ref, w2t_ref)):
            ew = ew_ref[...]
            sp = jnp.maximum(ew, 0.0) + jnp.log1p(jnp.exp(-jnp.abs(ew)))
            wt_ref[...] = jnp.broadcast_to(sp, (_B, 60))
    for c in range(_NCHUNK):
        pooled_ref[:, 128 * c:128 * (c + 1)] = jnp.dot(
            x_ref[:, _CHUNK * c:_CHUNK * (c + 1)], pc_ref[...],
            preferred_element_type=jnp.float32)
    pooled_ref[:, _NCHUNK * 128:(_NCHUNK + 1) * 128] = jnp.dot(
        x_ref[:, _NCHUNK * _CHUNK:_D0], pt_ref[...],
        preferred_element_type=jnp.float32)
    h = jnp.dot(pooled_ref[...], w2_ref[...], preferred_element_type=jnp.float32)
    h = jnp.maximum(h + b2_ref[...], 0.0)
    h1_ref[...] = h
    h3 = h.reshape(_R // _NEL, _NEL, 512)
    ps = jnp.sum(h3, axis=(0, 2)).reshape(1, _NEL)
    pq = jnp.sum(h3 * h3, axis=(0, 2)).reshape(1, _NEL)

    @pl.when(pl.program_id(0) == 0)
    def _init():
        s_ref[...] = jnp.zeros_like(s_ref)
        q_ref[...] = jnp.zeros_like(q_ref)

    s_ref[...] += ps
    q_ref[...] += pq


def _stage1(x, w2pad, b2, ew1, ew2):
    pc, pt = _pool_mats()
    return pl.pallas_call(
        _k1_body,
        grid=(_GRID,),
        in_specs=[
            pl.BlockSpec((_R, _D0), lambda i: (i, 0)),
            pl.BlockSpec((_CHUNK, 128), lambda i: (0, 0)),
            pl.BlockSpec((_D0 - _NCHUNK * _CHUNK, 128), lambda i: (0, 0)),
            pl.BlockSpec(((_NCHUNK + 1) * 128, 512), lambda i: (0, 0)),
            pl.BlockSpec((1, 512), lambda i: (0, 0)),
            pl.BlockSpec((1, 60), lambda i: (0, 0)),
            pl.BlockSpec((1, 60), lambda i: (0, 0)),
        ],
        out_specs=[
            pl.BlockSpec((_R, 512), lambda i: (i, 0)),
            pl.BlockSpec((1, _NEL), lambda i: (0, 0)),
            pl.BlockSpec((1, _NEL), lambda i: (0, 0)),
            pl.BlockSpec((_B, 60), lambda i: (0, 0)),
            pl.BlockSpec((_B, 60), lambda i: (0, 0)),
        ],
        out_shape=[
            jax.ShapeDtypeStruct((_N, 512), jnp.float32),
            jax.ShapeDtypeStruct((1, _NEL), jnp.float32),
            jax.ShapeDtypeStruct((1, _NEL), jnp.float32),
            jax.ShapeDtypeStruct((_B, 60), jnp.float32),
            jax.ShapeDtypeStruct((_B, 60), jnp.float32),
        ],
        scratch_shapes=[pltpu.VMEM((_R, (_NCHUNK + 1) * 128), jnp.float32)],
    )(x, pc, pt, w2pad, b2, ew1, ew2)


def _bn_scale_shift(s, q, count, g, be):
    m = s / count
    v = q / count - m * m
    inv = lax.rsqrt(v + 1e-5) * g
    return inv, be - m * inv


def _mk_mlp_bn(din, dout):
    def body(h_ref, sc_ref, sh_ref, w_ref, b_ref, out_ref, s_ref, q_ref):
        hb = h_ref[...] * sc_ref[...] + sh_ref[...]
        h2 = jnp.dot(hb, w_ref[...], preferred_element_type=jnp.float32)
        h2 = jnp.maximum(h2 + b_ref[...], 0.0)
        out_ref[...] = h2
        h3 = h2.reshape(_R // _NEL, _NEL, dout)
        ps = jnp.sum(h3, axis=(0, 2)).reshape(1, _NEL)
        pq = jnp.sum(h3 * h3, axis=(0, 2)).reshape(1, _NEL)

        @pl.when(pl.program_id(0) == 0)
        def _init():
            s_ref[...] = jnp.zeros_like(s_ref)
            q_ref[...] = jnp.zeros_like(q_ref)

        s_ref[...] += ps
        q_ref[...] += pq

    def call(h, sc_col, sh_col, w, b):
        return pl.pallas_call(
            body,
            grid=(_GRID,),
            in_specs=[
                pl.BlockSpec((_R, din), lambda i: (i, 0)),
                pl.BlockSpec((_R, 1), lambda i: (i, 0)),
                pl.BlockSpec((_R, 1), lambda i: (i, 0)),
                pl.BlockSpec((din, dout), lambda i: (0, 0)),
                pl.BlockSpec((1, dout), lambda i: (0, 0)),
            ],
            out_specs=[
                pl.BlockSpec((_R, dout), lambda i: (i, 0)),
                pl.BlockSpec((1, _NEL), lambda i: (0, 0)),
                pl.BlockSpec((1, _NEL), lambda i: (0, 0)),
            ],
            out_shape=[
                jax.ShapeDtypeStruct((_N, dout), jnp.float32),
                jax.ShapeDtypeStruct((1, _NEL), jnp.float32),
                jax.ShapeDtypeStruct((1, _NEL), jnp.float32),
            ],
        )(h, sc_col, sh_col, w, b)

    return call


def _mk_apply_root(din, dmid, col_scale, dpad=None):
    dpad = dpad or din

    def body(h_ref, sc_ref, sh_ref, wr_ref, hn_ref, root_ref):
        hn = h_ref[...] * sc_ref[...] + sh_ref[...]
        if dpad > din:
            hn_ref[:, :din] = hn
            hn_ref[:, din:] = jnp.zeros((_R, dpad - din), jnp.float32)
        else:
            hn_ref[...] = hn
        root_ref[...] = jnp.dot(hn, wr_ref[...], preferred_element_type=jnp.float32)

    sspec = (pl.BlockSpec((_R, 1), lambda i: (i, 0)) if col_scale
             else pl.BlockSpec((1, din), lambda i: (0, 0)))

    def call(h, sc, sh, wroot):
        return pl.pallas_call(
            body,
            grid=(_GRID,),
            in_specs=[
                pl.BlockSpec((_R, din), lambda i: (i, 0)),
                sspec, sspec,
                pl.BlockSpec((din, dmid), lambda i: (0, 0)),
            ],
            out_specs=[
                pl.BlockSpec((_R, dpad), lambda i: (i, 0)),
                pl.BlockSpec((_R, dmid), lambda i: (i, 0)),
            ],
            out_shape=[
                jax.ShapeDtypeStruct((_N, dpad), jnp.float32),
                jax.ShapeDtypeStruct((_N, dmid), jnp.float32),
            ],
        )(h, sc, sh, wroot)

    return call


def _mk_combine(din, dout):
    def body(a0_ref, a1_ref, root_ref, wrel_ref, brel_ref, out_ref, s_ref, q_ref):
        agg = a0_ref[...] + a1_ref[...]
        h = jnp.dot(agg, wrel_ref[...], preferred_element_type=jnp.float32)
        h = jnp.maximum(h + brel_ref[...] + root_ref[...], 0.0)
        out_ref[...] = h
        ps = jnp.sum(h, axis=0).reshape(1, dout)
        pq = jnp.sum(h * h, axis=0).reshape(1, dout)

        @pl.when(pl.program_id(0) == 0)
        def _init():
            s_ref[...] = jnp.zeros_like(s_ref)
            q_ref[...] = jnp.zeros_like(q_ref)

        s_ref[...] += ps
        q_ref[...] += pq

    def call(a0, a1, root, wrel, brel):
        return pl.pallas_call(
            body,
            grid=(_GRID,),
            in_specs=[
                pl.BlockSpec((_R, din), lambda i: (i, 0)),
                pl.BlockSpec((_R, din), lambda i: (i, 0)),
                pl.BlockSpec((_R, dout), lambda i: (i, 0)),
                pl.BlockSpec((din, dout), lambda i: (0, 0)),
                pl.BlockSpec((1, dout), lambda i: (0, 0)),
            ],
            out_specs=[
                pl.BlockSpec((_R, dout), lambda i: (i, 0)),
                pl.BlockSpec((1, dout), lambda i: (0, 0)),
                pl.BlockSpec((1, dout), lambda i: (0, 0)),
            ],
            out_shape=[
                jax.ShapeDtypeStruct((_N, dout), jnp.float32),
                jax.ShapeDtypeStruct((1, dout), jnp.float32),
                jax.ShapeDtypeStruct((1, dout), jnp.float32),
            ],
        )(a0, a1, root, wrel, brel)

    return call


_NEDGE_TOT = 15360
_NWORK = 32
_EPW = _NEDGE_TOT // _NWORK
_ECH = 96
_NCH = _EPW // _ECH
_RPW = _N // 16


def _mk_gconv_sc(d):
    mesh = plsc.VectorSubcoreMesh(core_axis_name="c", subcore_axis_name="s")

    @functools.partial(
        pl.kernel,
        out_type=jax.ShapeDtypeStruct((2, _N, d), jnp.float32),
        mesh=mesh,
        scratch_types=[
            pltpu.VMEM((_NCH, _ECH), jnp.int32),
            pltpu.VMEM((_NCH, _ECH), jnp.int32),
            pltpu.VMEM((_EPW,), jnp.float32),
            pltpu.VMEM((_EPW, d), jnp.float32),
            pltpu.VMEM_SHARED((_N, d), jnp.float32),
            pltpu.SemaphoreType.DMA,
        ],
    )
    def k(h_hbm, src_hbm, dst_hbm, w_hbm, z_hbm, out_hbm,
          sidx, didx, wv, rows, agg, sem):
        c = lax.axis_index("c")
        s = lax.axis_index("s")
        wid = s * 2 + c
        base = wid * _EPW
        pltpu.sync_copy(z_hbm.at[pl.ds(s * _RPW, _RPW)],
                        agg.at[pl.ds(s * _RPW, _RPW)])
        for j in range(_NCH):
            pltpu.sync_copy(src_hbm.at[pl.ds(base + _ECH * j, _ECH)], sidx.at[j])
            pltpu.sync_copy(dst_hbm.at[pl.ds(base + _ECH * j, _ECH)], didx.at[j])
        pltpu.sync_copy(w_hbm.at[pl.ds(base, _EPW)], wv)
        for j in range(_NCH):
            pltpu.async_copy(h_hbm.at[sidx.at[j]],
                             rows.at[pl.ds(_ECH * j, _ECH)], sem).wait()

        def scale(g, carry):
            wg = wv[pl.ds(g * 16, 16)]
            for i in range(16):
                e = g * 16 + i
                wvec = jnp.full((16,), 0.0, jnp.float32) + wg[i]
                for q in range(d // 16):
                    sl = pl.ds(q * 16, 16)
                    rows[e, sl] = rows[e, sl] * wvec
            return carry

        lax.fori_loop(0, _EPW // 16, scale, 0)
        plsc.subcore_barrier()
        for j in range(_NCH):
            pltpu.sync_copy(rows.at[pl.ds(_ECH * j, _ECH)],
                            agg.at[didx.at[j]], add=True)
        plsc.subcore_barrier()
        pltpu.sync_copy(agg.at[pl.ds(s * _RPW, _RPW)],
                        out_hbm.at[c].at[pl.ds(s * _RPW, _RPW)])

    return k


def _pool_body(h_ref, sc_ref, sh_ref, out_ref):
    hn = h_ref[...] * sc_ref[...] + sh_ref[...]
    out_ref[...] = jnp.max(hn.reshape(_R // _NEL, _NEL, 64), axis=1)


def _maxpool(h, sc, sh):
    return pl.pallas_call(
        _pool_body,
        grid=(_GRID,),
        in_specs=[
            pl.BlockSpec((_R, 64), lambda i: (i, 0)),
            pl.BlockSpec((1, 64), lambda i: (0, 0)),
            pl.BlockSpec((1, 64), lambda i: (0, 0)),
        ],
        out_specs=pl.BlockSpec((_R // _NEL, 64), lambda i: (i, 0)),
        out_shape=jax.ShapeDtypeStruct((_B, 64), jnp.float32),
    )(h, sc, sh)


def _heads_body(p_ref, w5_ref, b5_ref, wc_ref, bc_ref, out_ref):
    feat = jnp.dot(p_ref[...], w5_ref[...], preferred_element_type=jnp.float32)
    feat = jnp.maximum(feat + b5_ref[...], 0.0)
    z = jnp.dot(feat, wc_ref[...], preferred_element_type=jnp.float32) + bc_ref[...]
    ps = []
    for k in range(3):
        zp = z[:, 2 * k:2 * k + 2]
        m = jnp.max(zp, axis=1, keepdims=True)
        e = jnp.exp(zp - m)
        ps.append(e / jnp.sum(e, axis=1, keepdims=True))
    p0, p1, p2 = ps
    p_hc = p0[:, 0:1] * p1[:, 0:1]
    p_ad = p0[:, 1:2] * p2[:, 1:2]
    p_ftd = p0[:, 0:1] * p1[:, 1:2] + p0[:, 1:2] * p2[:, 0:1]
    out_ref[...] = jnp.log(jnp.concatenate([p_hc, p_ftd, p_ad], axis=1) + 1e-8)


def _heads(pooled, w5, b5, wcat, bcat):
    return pl.pallas_call(
        _heads_body,
        in_specs=[pl.BlockSpec(pooled.shape, lambda: (0, 0))] * 1 + [
            pl.BlockSpec(w5.shape, lambda: (0, 0)),
            pl.BlockSpec(b5.shape, lambda: (0, 0)),
            pl.BlockSpec(wcat.shape, lambda: (0, 0)),
            pl.BlockSpec(bcat.shape, lambda: (0, 0)),
        ],
        out_specs=pl.BlockSpec((_B, 3), lambda: (0, 0)),
        out_shape=jax.ShapeDtypeStruct((_B, 3), jnp.float32),
    )(pooled, w5, b5, wcat, bcat)


def kernel(x, edge_index, batch, W2, b2, g3, be3, W3, b3, g4, be4, W4, b4,
           g5, be5, ew1, Wrel1, brel1, Wroot1, g6, be6, ew2, Wrel2, brel2,
           Wroot2, g7, be7, W5, b5, Whr, bhr, Whf, bhf, Wfa, bfa):
    w2pad = jnp.concatenate(
        [W2, jnp.zeros(((_NCHUNK + 1) * 128 - _NPOOL, 512), jnp.float32)], axis=0)
    h1, s1, q1, w1t, w2t = _stage1(x, w2pad, b2.reshape(1, 512),
                                   ew1.reshape(1, 60), ew2.reshape(1, 60))

    def col(v19):
        return jnp.tile(v19, _B)[:, None]

    inv1, sh1 = _bn_scale_shift(s1[0], q1[0], _B * 512.0, g3, be3)
    h2, s2, q2 = _mk_mlp_bn(512, 256)(h1, col(inv1), col(sh1), W3,
                                      b3.reshape(1, 256))
    inv2, sh2 = _bn_scale_shift(s2[0], q2[0], _B * 256.0, g4, be4)
    h3, s3, q3 = _mk_mlp_bn(256, 128)(h2, col(inv2), col(sh2), W4,
                                      b4.reshape(1, 128))
    inv3, sh3 = _bn_scale_shift(s3[0], q3[0], _B * 128.0, g5, be5)
    h3n, root1 = _mk_apply_root(128, 64, True)(h3, col(inv3), col(sh3), Wroot1)

    src = edge_index[0]
    dst = edge_index[1]
    w1e = w1t.reshape(_NEDGE_TOT)
    w2e = w2t.reshape(_NEDGE_TOT)

    z128 = jnp.zeros((_N, 128), jnp.float32)
    parts1 = _mk_gconv_sc(128)(h3n, src, dst, w1e, z128)
    h4, s4, q4 = _mk_combine(128, 64)(parts1[0], parts1[1], root1, Wrel1,
                                      brel1.reshape(1, 64))
    inv4, sh4 = _bn_scale_shift(s4[0:1], q4[0:1], float(_N), g6.reshape(1, 64),
                                be6.reshape(1, 64))
    h4n, root2 = _mk_apply_root(64, 64, False, dpad=128)(h4, inv4, sh4, Wroot2)

    parts2 = _mk_gconv_sc(128)(h4n, src, dst, w2e, z128)
    wrel2p = jnp.concatenate([Wrel2, jnp.zeros((64, 64), jnp.float32)], axis=0)
    h5, s5, q5 = _mk_combine(128, 64)(parts2[0], parts2[1], root2, wrel2p,
                                      brel2.reshape(1, 64))
    inv5, sh5 = _bn_scale_shift(s5[0:1], q5[0:1], float(_N), g7.reshape(1, 64),
                                be7.reshape(1, 64))
    pooled = _maxpool(h5, inv5, sh5)
    wcat = jnp.concatenate([Whr, Whf, Wfa], axis=1)
    bcat = jnp.concatenate([bhr, bhf, bfa]).reshape(1, 6)
    return _heads(pooled, W5, b5.reshape(1, 32), wcat, bcat)


# speedup vs baseline: 2.3330x; 1.8046x over previous
import functools

import jax
import jax.numpy as jnp
from jax import lax
from jax.experimental import pallas as pl
from jax.experimental.pallas import tpu as pltpu
from jax.experimental.pallas import tpu_sc as plsc

_B = 256
_NEL = 19
_N = _B * _NEL
_D0 = 20000
_WLEN = 25
_NPOOL = 800
_CHUNK = 3200
_NCHUNK = 6
_R = 152
_GRID = _N // _R


def _pool_mats():
    j = jnp.arange(_CHUNK)
    pc = (j[:, None] // _WLEN == jnp.arange(128)[None, :]).astype(jnp.float32) / _WLEN
    jt = jnp.arange(_D0 - _NCHUNK * _CHUNK)
    pt = (jt[:, None] // _WLEN == jnp.arange(128)[None, :]).astype(jnp.float32) / _WLEN
    return pc, pt


def _k1a_body(xt_ref, pc_ref, pt_ref, w2_ref, b2_ref, h1_ref, pooled_ref):
    dn = (((0,), (0,)), ((), ()))
    for c in range(_NCHUNK):
        pooled_ref[:, 128 * c:128 * (c + 1)] = lax.dot_general(
            xt_ref[pl.ds(_CHUNK * c, _CHUNK), :], pc_ref[...], dn,
            preferred_element_type=jnp.float32)
    pooled_ref[:, _NCHUNK * 128:(_NCHUNK + 1) * 128] = lax.dot_general(
        xt_ref[pl.ds(_NCHUNK * _CHUNK, _D0 - _NCHUNK * _CHUNK), :], pt_ref[...],
        dn, preferred_element_type=jnp.float32)
    h = jnp.dot(pooled_ref[...], w2_ref[...], preferred_element_type=jnp.float32)
    h1_ref[...] = jnp.maximum(h + b2_ref[...], 0.0)


def _stage1(xt, w2pad, b2):
    pc, pt = _pool_mats()
    return pl.pallas_call(
        _k1a_body,
        grid=(_N // 128,),
        in_specs=[
            pl.BlockSpec((_D0, 128), lambda i: (0, i)),
            pl.BlockSpec((_CHUNK, 128), lambda i: (0, 0)),
            pl.BlockSpec((_D0 - _NCHUNK * _CHUNK, 128), lambda i: (0, 0)),
            pl.BlockSpec(((_NCHUNK + 1) * 128, 512), lambda i: (0, 0)),
            pl.BlockSpec((1, 512), lambda i: (0, 0)),
        ],
        out_specs=pl.BlockSpec((128, 512), lambda i: (i, 0)),
        out_shape=jax.ShapeDtypeStruct((_N, 512), jnp.float32),
        scratch_shapes=[pltpu.VMEM((128, (_NCHUNK + 1) * 128), jnp.float32)],
    )(xt, pc, pt, w2pad, b2)


def _k1s_body(h_ref, ew1_ref, ew2_ref, s_ref, q_ref, w1t_ref, w2t_ref):
    @pl.when(pl.program_id(0) == 0)
    def _wexp():
        for ew_ref, wt_ref in ((ew1_ref, w1t_ref), (ew2_ref, w2t_ref)):
            ew = ew_ref[...]
            sp = jnp.maximum(ew, 0.0) + jnp.log1p(jnp.exp(-jnp.abs(ew)))
            wt_ref[...] = jnp.broadcast_to(sp, (_B, 60))

    h = h_ref[...]
    rs = jnp.sum(h, axis=1)
    rq = jnp.sum(h * h, axis=1)
    ps = jnp.sum(rs.reshape(_R // _NEL, _NEL), axis=0).reshape(1, _NEL)
    pq = jnp.sum(rq.reshape(_R // _NEL, _NEL), axis=0).reshape(1, _NEL)

    @pl.when(pl.program_id(0) == 0)
    def _init():
        s_ref[...] = jnp.zeros_like(s_ref)
        q_ref[...] = jnp.zeros_like(q_ref)

    s_ref[...] += ps
    q_ref[...] += pq


def _stage1_stats(h1, ew1, ew2):
    return pl.pallas_call(
        _k1s_body,
        grid=(_GRID,),
        in_specs=[
            pl.BlockSpec((_R, 512), lambda i: (i, 0)),
            pl.BlockSpec((1, 60), lambda i: (0, 0)),
            pl.BlockSpec((1, 60), lambda i: (0, 0)),
        ],
        out_specs=[
            pl.BlockSpec((1, _NEL), lambda i: (0, 0)),
            pl.BlockSpec((1, _NEL), lambda i: (0, 0)),
            pl.BlockSpec((_B, 60), lambda i: (0, 0)),
            pl.BlockSpec((_B, 60), lambda i: (0, 0)),
        ],
        out_shape=[
            jax.ShapeDtypeStruct((1, _NEL), jnp.float32),
            jax.ShapeDtypeStruct((1, _NEL), jnp.float32),
            jax.ShapeDtypeStruct((_B, 60), jnp.float32),
            jax.ShapeDtypeStruct((_B, 60), jnp.float32),
        ],
    )(h1, ew1, ew2)


def _bn_scale_shift(s, q, count, g, be):
    m = s / count
    v = q / count - m * m
    inv = lax.rsqrt(v + 1e-5) * g
    return inv, be - m * inv


def _mk_mlp_bn(din, dout):
    def body(h_ref, sc_ref, sh_ref, w_ref, b_ref, out_ref, s_ref, q_ref):
        hb = h_ref[...] * sc_ref[...] + sh_ref[...]
        h2 = jnp.dot(hb, w_ref[...], preferred_element_type=jnp.float32)
        h2 = jnp.maximum(h2 + b_ref[...], 0.0)
        out_ref[...] = h2
        h3 = h2.reshape(_R // _NEL, _NEL, dout)
        ps = jnp.sum(h3, axis=(0, 2)).reshape(1, _NEL)
        pq = jnp.sum(h3 * h3, axis=(0, 2)).reshape(1, _NEL)

        @pl.when(pl.program_id(0) == 0)
        def _init():
            s_ref[...] = jnp.zeros_like(s_ref)
            q_ref[...] = jnp.zeros_like(q_ref)

        s_ref[...] += ps
        q_ref[...] += pq

    def call(h, sc_col, sh_col, w, b):
        return pl.pallas_call(
            body,
            grid=(_GRID,),
            in_specs=[
                pl.BlockSpec((_R, din), lambda i: (i, 0)),
                pl.BlockSpec((_R, 1), lambda i: (i, 0)),
                pl.BlockSpec((_R, 1), lambda i: (i, 0)),
                pl.BlockSpec((din, dout), lambda i: (0, 0)),
                pl.BlockSpec((1, dout), lambda i: (0, 0)),
            ],
            out_specs=[
                pl.BlockSpec((_R, dout), lambda i: (i, 0)),
                pl.BlockSpec((1, _NEL), lambda i: (0, 0)),
                pl.BlockSpec((1, _NEL), lambda i: (0, 0)),
            ],
            out_shape=[
                jax.ShapeDtypeStruct((_N, dout), jnp.float32),
                jax.ShapeDtypeStruct((1, _NEL), jnp.float32),
                jax.ShapeDtypeStruct((1, _NEL), jnp.float32),
            ],
        )(h, sc_col, sh_col, w, b)

    return call


def _mk_apply_root(din, dmid, col_scale, dpad=None):
    dpad = dpad or din

    def body(h_ref, sc_ref, sh_ref, wr_ref, hn_ref, root_ref):
        hn = h_ref[...] * sc_ref[...] + sh_ref[...]
        if dpad > din:
            hn_ref[:, :din] = hn
            hn_ref[:, din:] = jnp.zeros((_R, dpad - din), jnp.float32)
        else:
            hn_ref[...] = hn
        root_ref[...] = jnp.dot(hn, wr_ref[...], preferred_element_type=jnp.float32)

    sspec = (pl.BlockSpec((_R, 1), lambda i: (i, 0)) if col_scale
             else pl.BlockSpec((1, din), lambda i: (0, 0)))

    def call(h, sc, sh, wroot):
        return pl.pallas_call(
            body,
            grid=(_GRID,),
            in_specs=[
                pl.BlockSpec((_R, din), lambda i: (i, 0)),
                sspec, sspec,
                pl.BlockSpec((din, dmid), lambda i: (0, 0)),
            ],
            out_specs=[
                pl.BlockSpec((_R, dpad), lambda i: (i, 0)),
                pl.BlockSpec((_R, dmid), lambda i: (i, 0)),
            ],
            out_shape=[
                jax.ShapeDtypeStruct((_N, dpad), jnp.float32),
                jax.ShapeDtypeStruct((_N, dmid), jnp.float32),
            ],
        )(h, sc, sh, wroot)

    return call


def _mk_combine(din, dout):
    def body(a0_ref, a1_ref, root_ref, wrel_ref, brel_ref, out_ref, s_ref, q_ref):
        agg = a0_ref[...] + a1_ref[...]
        h = jnp.dot(agg, wrel_ref[...], preferred_element_type=jnp.float32)
        h = jnp.maximum(h + brel_ref[...] + root_ref[...], 0.0)
        out_ref[...] = h
        ps = jnp.sum(h, axis=0).reshape(1, dout)
        pq = jnp.sum(h * h, axis=0).reshape(1, dout)

        @pl.when(pl.program_id(0) == 0)
        def _init():
            s_ref[...] = jnp.zeros_like(s_ref)
            q_ref[...] = jnp.zeros_like(q_ref)

        s_ref[...] += ps
        q_ref[...] += pq

    def call(a0, a1, root, wrel, brel):
        return pl.pallas_call(
            body,
            grid=(_GRID,),
            in_specs=[
                pl.BlockSpec((_R, din), lambda i: (i, 0)),
                pl.BlockSpec((_R, din), lambda i: (i, 0)),
                pl.BlockSpec((_R, dout), lambda i: (i, 0)),
                pl.BlockSpec((din, dout), lambda i: (0, 0)),
                pl.BlockSpec((1, dout), lambda i: (0, 0)),
            ],
            out_specs=[
                pl.BlockSpec((_R, dout), lambda i: (i, 0)),
                pl.BlockSpec((1, dout), lambda i: (0, 0)),
                pl.BlockSpec((1, dout), lambda i: (0, 0)),
            ],
            out_shape=[
                jax.ShapeDtypeStruct((_N, dout), jnp.float32),
                jax.ShapeDtypeStruct((1, dout), jnp.float32),
                jax.ShapeDtypeStruct((1, dout), jnp.float32),
            ],
        )(a0, a1, root, wrel, brel)

    return call


_NEDGE_TOT = 15360
_NWORK = 32
_EPW = _NEDGE_TOT // _NWORK
_ECH = 96
_NCH = _EPW // _ECH
_RPW = _N // 16


def _mk_gconv_sc(d):
    mesh = plsc.VectorSubcoreMesh(core_axis_name="c", subcore_axis_name="s")

    @functools.partial(
        pl.kernel,
        out_type=jax.ShapeDtypeStruct((2, _N, d), jnp.float32),
        mesh=mesh,
        scratch_types=[
            pltpu.VMEM((_NCH, _ECH), jnp.int32),
            pltpu.VMEM((_NCH, _ECH), jnp.int32),
            pltpu.VMEM((_EPW,), jnp.float32),
            pltpu.VMEM((_EPW, d), jnp.float32),
            pltpu.VMEM_SHARED((_N, d), jnp.float32),
            pltpu.SemaphoreType.DMA,
        ],
    )
    def k(h_hbm, src_hbm, dst_hbm, w_hbm, z_hbm, out_hbm,
          sidx, didx, wv, rows, agg, sem):
        c = lax.axis_index("c")
        s = lax.axis_index("s")
        wid = s * 2 + c
        base = wid * _EPW
        pltpu.sync_copy(z_hbm.at[pl.ds(s * _RPW, _RPW)],
                        agg.at[pl.ds(s * _RPW, _RPW)])
        for j in range(_NCH):
            pltpu.sync_copy(src_hbm.at[pl.ds(base + _ECH * j, _ECH)], sidx.at[j])
            pltpu.sync_copy(dst_hbm.at[pl.ds(base + _ECH * j, _ECH)], didx.at[j])
        pltpu.sync_copy(w_hbm.at[pl.ds(base, _EPW)], wv)
        for j in range(_NCH):
            pltpu.async_copy(h_hbm.at[sidx.at[j]],
                             rows.at[pl.ds(_ECH * j, _ECH)], sem).wait()

        def scale(g, carry):
            wg = wv[pl.ds(g * 16, 16)]
            for i in range(16):
                e = g * 16 + i
                wvec = jnp.full((16,), 0.0, jnp.float32) + wg[i]
                for q in range(d // 16):
                    sl = pl.ds(q * 16, 16)
                    rows[e, sl] = rows[e, sl] * wvec
            return carry

        lax.fori_loop(0, _EPW // 16, scale, 0)
        plsc.subcore_barrier()
        for j in range(_NCH):
            pltpu.sync_copy(rows.at[pl.ds(_ECH * j, _ECH)],
                            agg.at[didx.at[j]], add=True)
        plsc.subcore_barrier()
        pltpu.sync_copy(agg.at[pl.ds(s * _RPW, _RPW)],
                        out_hbm.at[c].at[pl.ds(s * _RPW, _RPW)])

    return k


def _pool_body(h_ref, sc_ref, sh_ref, out_ref):
    hn = h_ref[...] * sc_ref[...] + sh_ref[...]
    out_ref[...] = jnp.max(hn.reshape(_R // _NEL, _NEL, 64), axis=1)


def _maxpool(h, sc, sh):
    return pl.pallas_call(
        _pool_body,
        grid=(_GRID,),
        in_specs=[
            pl.BlockSpec((_R, 64), lambda i: (i, 0)),
            pl.BlockSpec((1, 64), lambda i: (0, 0)),
            pl.BlockSpec((1, 64), lambda i: (0, 0)),
        ],
        out_specs=pl.BlockSpec((_R // _NEL, 64), lambda i: (i, 0)),
        out_shape=jax.ShapeDtypeStruct((_B, 64), jnp.float32),
    )(h, sc, sh)


def _heads_body(p_ref, w5_ref, b5_ref, wc_ref, bc_ref, out_ref):
    feat = jnp.dot(p_ref[...], w5_ref[...], preferred_element_type=jnp.float32)
    feat = jnp.maximum(feat + b5_ref[...], 0.0)
    z = jnp.dot(feat, wc_ref[...], preferred_element_type=jnp.float32) + bc_ref[...]
    ps = []
    for k in range(3):
        zp = z[:, 2 * k:2 * k + 2]
        m = jnp.max(zp, axis=1, keepdims=True)
        e = jnp.exp(zp - m)
        ps.append(e / jnp.sum(e, axis=1, keepdims=True))
    p0, p1, p2 = ps
    p_hc = p0[:, 0:1] * p1[:, 0:1]
    p_ad = p0[:, 1:2] * p2[:, 1:2]
    p_ftd = p0[:, 0:1] * p1[:, 1:2] + p0[:, 1:2] * p2[:, 0:1]
    out_ref[...] = jnp.log(jnp.concatenate([p_hc, p_ftd, p_ad], axis=1) + 1e-8)


def _heads(pooled, w5, b5, wcat, bcat):
    return pl.pallas_call(
        _heads_body,
        in_specs=[pl.BlockSpec(pooled.shape, lambda: (0, 0))] * 1 + [
            pl.BlockSpec(w5.shape, lambda: (0, 0)),
            pl.BlockSpec(b5.shape, lambda: (0, 0)),
            pl.BlockSpec(wcat.shape, lambda: (0, 0)),
            pl.BlockSpec(bcat.shape, lambda: (0, 0)),
        ],
        out_specs=pl.BlockSpec((_B, 3), lambda: (0, 0)),
        out_shape=jax.ShapeDtypeStruct((_B, 3), jnp.float32),
    )(pooled, w5, b5, wcat, bcat)


def kernel(x, edge_index, batch, W2, b2, g3, be3, W3, b3, g4, be4, W4, b4,
           g5, be5, ew1, Wrel1, brel1, Wroot1, g6, be6, ew2, Wrel2, brel2,
           Wroot2, g7, be7, W5, b5, Whr, bhr, Whf, bhf, Wfa, bfa):
    w2pad = jnp.concatenate(
        [W2, jnp.zeros(((_NCHUNK + 1) * 128 - _NPOOL, 512), jnp.float32)], axis=0)
    h1 = _stage1(jnp.swapaxes(x, 0, 1), w2pad, b2.reshape(1, 512))
    s1, q1, w1t, w2t = _stage1_stats(h1, ew1.reshape(1, 60), ew2.reshape(1, 60))

    def col(v19):
        return jnp.tile(v19, _B)[:, None]

    inv1, sh1 = _bn_scale_shift(s1[0], q1[0], _B * 512.0, g3, be3)
    h2, s2, q2 = _mk_mlp_bn(512, 256)(h1, col(inv1), col(sh1), W3,
                                      b3.reshape(1, 256))
    inv2, sh2 = _bn_scale_shift(s2[0], q2[0], _B * 256.0, g4, be4)
    h3, s3, q3 = _mk_mlp_bn(256, 128)(h2, col(inv2), col(sh2), W4,
                                      b4.reshape(1, 128))
    inv3, sh3 = _bn_scale_shift(s3[0], q3[0], _B * 128.0, g5, be5)
    h3n, root1 = _mk_apply_root(128, 64, True)(h3, col(inv3), col(sh3), Wroot1)

    src = edge_index[0]
    dst = edge_index[1]
    w1e = w1t.reshape(_NEDGE_TOT)
    w2e = w2t.reshape(_NEDGE_TOT)

    z128 = jnp.zeros((_N, 128), jnp.float32)
    parts1 = _mk_gconv_sc(128)(h3n, src, dst, w1e, z128)
    h4, s4, q4 = _mk_combine(128, 64)(parts1[0], parts1[1], root1, Wrel1,
                                      brel1.reshape(1, 64))
    inv4, sh4 = _bn_scale_shift(s4[0:1], q4[0:1], float(_N), g6.reshape(1, 64),
                                be6.reshape(1, 64))
    h4n, root2 = _mk_apply_root(64, 64, False, dpad=128)(h4, inv4, sh4, Wroot2)

    parts2 = _mk_gconv_sc(128)(h4n, src, dst, w2e, z128)
    wrel2p = jnp.concatenate([Wrel2, jnp.zeros((64, 64), jnp.float32)], axis=0)
    h5, s5, q5 = _mk_combine(128, 64)(parts2[0], parts2[1], root2, wrel2p,
                                      brel2.reshape(1, 64))
    inv5, sh5 = _bn_scale_shift(s5[0:1], q5[0:1], float(_N), g7.reshape(1, 64),
                                be7.reshape(1, 64))
    pooled = _maxpool(h5, inv5, sh5)
    wcat = jnp.concatenate([Whr, Whf, Wfa], axis=1)
    bcat = jnp.concatenate([bhr, bhf, bfa]).reshape(1, 6)
    return _heads(pooled, W5, b5.reshape(1, 32), wcat, bcat)


# speedup vs baseline: 3.2508x; 1.3934x over previous
import functools

import jax
import jax.numpy as jnp
from jax import lax
from jax.experimental import pallas as pl
from jax.experimental.pallas import tpu as pltpu
from jax.experimental.pallas import tpu_sc as plsc

_B = 256
_NEL = 19
_N = _B * _NEL
_D0 = 20000
_WLEN = 25
_NPOOL = 800
_CHUNK = 3200
_NCHUNK = 6
_RB = 608
_G8 = _N // _RB


def _pool_mats():
    j = jnp.arange(_CHUNK)
    pc = (j[:, None] // _WLEN == jnp.arange(128)[None, :]).astype(jnp.float32) / _WLEN
    jt = jnp.arange(_D0 - _NCHUNK * _CHUNK)
    pt = (jt[:, None] // _WLEN == jnp.arange(128)[None, :]).astype(jnp.float32) / _WLEN
    return pc, pt


def _onehot_el():
    return (jnp.arange(_N)[:, None] % _NEL == jnp.arange(_NEL)[None, :]
            ).astype(jnp.float32)


def _k1a_body(xt_ref, pc_ref, pt_ref, w2_ref, b2_ref,
              h1_ref, rs_ref, rq_ref, pooled_ref):
    dn = (((0,), (0,)), ((), ()))
    for c in range(_NCHUNK):
        pooled_ref[:, 128 * c:128 * (c + 1)] = lax.dot_general(
            xt_ref[pl.ds(_CHUNK * c, _CHUNK), :], pc_ref[...], dn,
            preferred_element_type=jnp.float32)
    pooled_ref[:, _NCHUNK * 128:(_NCHUNK + 1) * 128] = lax.dot_general(
        xt_ref[pl.ds(_NCHUNK * _CHUNK, _D0 - _NCHUNK * _CHUNK), :], pt_ref[...],
        dn, preferred_element_type=jnp.float32)
    h = jnp.dot(pooled_ref[...], w2_ref[...], preferred_element_type=jnp.float32)
    h = jnp.maximum(h + b2_ref[...], 0.0)
    h1_ref[...] = h
    rs_ref[...] = jnp.sum(h, axis=1, keepdims=True)
    rq_ref[...] = jnp.sum(h * h, axis=1, keepdims=True)


def _stage1(xt, w2pad, b2):
    pc, pt = _pool_mats()
    nblk = _N // 128
    return pl.pallas_call(
        _k1a_body,
        grid=(nblk,),
        in_specs=[
            pl.BlockSpec((_D0, 128), lambda i: (0, i)),
            pl.BlockSpec((_CHUNK, 128), lambda i: (0, 0)),
            pl.BlockSpec((_D0 - _NCHUNK * _CHUNK, 128), lambda i: (0, 0)),
            pl.BlockSpec(((_NCHUNK + 1) * 128, 512), lambda i: (0, 0)),
            pl.BlockSpec((1, 512), lambda i: (0, 0)),
        ],
        out_specs=[
            pl.BlockSpec((128, 512), lambda i: (i, 0)),
            pl.BlockSpec((128, 1), lambda i: (i, 0)),
            pl.BlockSpec((128, 1), lambda i: (i, 0)),
        ],
        out_shape=[
            jax.ShapeDtypeStruct((_N, 512), jnp.float32),
            jax.ShapeDtypeStruct((_N, 1), jnp.float32),
            jax.ShapeDtypeStruct((_N, 1), jnp.float32),
        ],
        scratch_shapes=[pltpu.VMEM((128, (_NCHUNK + 1) * 128), jnp.float32)],
    )(xt, pc, pt, w2pad, b2)


def _mk_bn_resolve(rshape, count, with_wexp):
    rows, cols = rshape

    def body(*refs):
        if with_wexp:
            (rs_ref, rq_ref, oh_ref, g_ref, be_ref, ew1_ref, ew2_ref,
             sc_ref, sh_ref, w1t_ref, w2t_ref) = refs
            for ew_ref, wt_ref in ((ew1_ref, w1t_ref), (ew2_ref, w2t_ref)):
                ew = ew_ref[...]
                sp = jnp.maximum(ew, 0.0) + jnp.log1p(jnp.exp(-jnp.abs(ew)))
                wt_ref[...] = jnp.broadcast_to(sp, (_B, 60))
        else:
            rs_ref, rq_ref, oh_ref, g_ref, be_ref, sc_ref, sh_ref = refs
        oh = oh_ref[...]
        dn0 = (((0,), (0,)), ((), ()))
        s19 = lax.dot_general(rs_ref[...], oh, dn0,
                              preferred_element_type=jnp.float32)
        q19 = lax.dot_general(rq_ref[...], oh, dn0,
                              preferred_element_type=jnp.float32)
        m = s19 / count
        v = q19 / count - m * m
        inv = lax.rsqrt(v + 1e-5) * g_ref[...]
        sh = be_ref[...] - m * inv
        dn = (((1,), (1,)), ((), ()))
        sc_ref[...] = lax.dot_general(oh, inv, dn,
                                      preferred_element_type=jnp.float32)
        sh_ref[...] = lax.dot_general(oh, sh, dn,
                                      preferred_element_type=jnp.float32)

    out_shape = [
        jax.ShapeDtypeStruct((_N, 1), jnp.float32),
        jax.ShapeDtypeStruct((_N, 1), jnp.float32),
    ]
    if with_wexp:
        out_shape += [jax.ShapeDtypeStruct((_B, 60), jnp.float32)] * 2

    def call(rs, rq, g, be, *ews):
        return pl.pallas_call(body, out_shape=out_shape)(
            rs, rq, _onehot_el(), g.reshape(1, _NEL), be.reshape(1, _NEL), *ews)

    return call


def _mk_mlp_bn(din, dout):
    def body(h_ref, sc_ref, sh_ref, w_ref, b_ref, out_ref, rs_ref, rq_ref):
        hb = h_ref[...] * sc_ref[...] + sh_ref[...]
        h2 = jnp.dot(hb, w_ref[...], preferred_element_type=jnp.float32)
        h2 = jnp.maximum(h2 + b_ref[...], 0.0)
        out_ref[...] = h2
        rs_ref[...] = jnp.sum(h2, axis=1, keepdims=True)
        rq_ref[...] = jnp.sum(h2 * h2, axis=1, keepdims=True)

    def call(h, sc_col, sh_col, w, b):
        return pl.pallas_call(
            body,
            grid=(_G8,),
            in_specs=[
                pl.BlockSpec((_RB, din), lambda i: (i, 0)),
                pl.BlockSpec((_RB, 1), lambda i: (i, 0)),
                pl.BlockSpec((_RB, 1), lambda i: (i, 0)),
                pl.BlockSpec((din, dout), lambda i: (0, 0)),
                pl.BlockSpec((1, dout), lambda i: (0, 0)),
            ],
            out_specs=[
                pl.BlockSpec((_RB, dout), lambda i: (i, 0)),
                pl.BlockSpec((_RB, 1), lambda i: (i, 0)),
                pl.BlockSpec((_RB, 1), lambda i: (i, 0)),
            ],
            out_shape=[
                jax.ShapeDtypeStruct((_N, dout), jnp.float32),
                jax.ShapeDtypeStruct((_N, 1), jnp.float32),
                jax.ShapeDtypeStruct((_N, 1), jnp.float32),
            ],
        )(h, sc_col, sh_col, w, b)

    return call


def _mk_apply_root(din, dmid, col_scale, dpad=None):
    dpad = dpad or din

    def body(h_ref, a_ref, b_ref, g_ref, be_ref, wr_ref, hn_ref, root_ref):
        if col_scale:
            sc, sh = a_ref[...], b_ref[...]
        else:
            m = a_ref[...] / float(_N)
            v = b_ref[...] / float(_N) - m * m
            sc = lax.rsqrt(v + 1e-5) * g_ref[...]
            sh = be_ref[...] - m * sc
        hn = h_ref[...] * sc + sh
        if dpad > din:
            hn_ref[:, :din] = hn
            hn_ref[:, din:] = jnp.zeros((_RB, dpad - din), jnp.float32)
        else:
            hn_ref[...] = hn
        root_ref[...] = jnp.dot(hn, wr_ref[...], preferred_element_type=jnp.float32)

    sspec = (pl.BlockSpec((_RB, 1), lambda i: (i, 0)) if col_scale
             else pl.BlockSpec((1, din), lambda i: (0, 0)))
    gspec = pl.BlockSpec((1, din), lambda i: (0, 0))

    def call(h, a, b, g, be, wroot):
        return pl.pallas_call(
            body,
            grid=(_G8,),
            in_specs=[
                pl.BlockSpec((_RB, din), lambda i: (i, 0)),
                sspec, sspec, gspec, gspec,
                pl.BlockSpec((din, dmid), lambda i: (0, 0)),
            ],
            out_specs=[
                pl.BlockSpec((_RB, dpad), lambda i: (i, 0)),
                pl.BlockSpec((_RB, dmid), lambda i: (i, 0)),
            ],
            out_shape=[
                jax.ShapeDtypeStruct((_N, dpad), jnp.float32),
                jax.ShapeDtypeStruct((_N, dmid), jnp.float32),
            ],
        )(h, a, b, g, be, wroot)

    return call


def _mk_combine(din, dout):
    def body(p_ref, root_ref, wrel_ref, brel_ref, out_ref, s_ref, q_ref):
        agg = p_ref[0] + p_ref[1]
        h = jnp.dot(agg, wrel_ref[...], preferred_element_type=jnp.float32)
        h = jnp.maximum(h + brel_ref[...] + root_ref[...], 0.0)
        out_ref[...] = h
        ps = jnp.sum(h, axis=0).reshape(1, dout)
        pq = jnp.sum(h * h, axis=0).reshape(1, dout)

        @pl.when(pl.program_id(0) == 0)
        def _init():
            s_ref[...] = jnp.zeros_like(s_ref)
            q_ref[...] = jnp.zeros_like(q_ref)

        s_ref[...] += ps
        q_ref[...] += pq

    def call(parts, root, wrel, brel):
        return pl.pallas_call(
            body,
            grid=(_G8,),
            in_specs=[
                pl.BlockSpec((2, _RB, din), lambda i: (0, i, 0)),
                pl.BlockSpec((_RB, dout), lambda i: (i, 0)),
                pl.BlockSpec((din, dout), lambda i: (0, 0)),
                pl.BlockSpec((1, dout), lambda i: (0, 0)),
            ],
            out_specs=[
                pl.BlockSpec((_RB, dout), lambda i: (i, 0)),
                pl.BlockSpec((1, dout), lambda i: (0, 0)),
                pl.BlockSpec((1, dout), lambda i: (0, 0)),
            ],
            out_shape=[
                jax.ShapeDtypeStruct((_N, dout), jnp.float32),
                jax.ShapeDtypeStruct((1, dout), jnp.float32),
                jax.ShapeDtypeStruct((1, dout), jnp.float32),
            ],
        )(parts, root, wrel, brel)

    return call


_NEDGE_TOT = 15360
_NWORK = 32
_EPW = _NEDGE_TOT // _NWORK
_ECH = 96
_NCH = _EPW // _ECH
_RPW = _N // 16


def _mk_gconv_sc(d):
    mesh = plsc.VectorSubcoreMesh(core_axis_name="c", subcore_axis_name="s")

    @functools.partial(
        pl.kernel,
        out_type=jax.ShapeDtypeStruct((2, _N, d), jnp.float32),
        mesh=mesh,
        scratch_types=[
            pltpu.VMEM((_NCH, _ECH), jnp.int32),
            pltpu.VMEM((_NCH, _ECH), jnp.int32),
            pltpu.VMEM((_EPW,), jnp.float32),
            pltpu.VMEM((_EPW, d), jnp.float32),
            pltpu.VMEM_SHARED((_N, d), jnp.float32),
            pltpu.SemaphoreType.DMA,
        ],
    )
    def k(h_hbm, src_hbm, dst_hbm, w_hbm, z_hbm, out_hbm,
          sidx, didx, wv, rows, agg, sem):
        c = lax.axis_index("c")
        s = lax.axis_index("s")
        wid = s * 2 + c
        base = wid * _EPW
        pltpu.sync_copy(z_hbm.at[pl.ds(s * _RPW, _RPW)],
                        agg.at[pl.ds(s * _RPW, _RPW)])
        for j in range(_NCH):
            pltpu.sync_copy(src_hbm.at[pl.ds(base + _ECH * j, _ECH)], sidx.at[j])
            pltpu.sync_copy(dst_hbm.at[pl.ds(base + _ECH * j, _ECH)], didx.at[j])
        pltpu.sync_copy(w_hbm.at[pl.ds(base, _EPW)], wv)
        for j in range(_NCH):
            pltpu.async_copy(h_hbm.at[sidx.at[j]],
                             rows.at[pl.ds(_ECH * j, _ECH)], sem).wait()

        def scale(g, carry):
            wg = wv[pl.ds(g * 16, 16)]
            for i in range(16):
                e = g * 16 + i
                wvec = jnp.full((16,), 0.0, jnp.float32) + wg[i]
                for q in range(d // 16):
                    sl = pl.ds(q * 16, 16)
                    rows[e, sl] = rows[e, sl] * wvec
            return carry

        lax.fori_loop(0, _EPW // 16, scale, 0)
        plsc.subcore_barrier()
        for j in range(_NCH):
            pltpu.sync_copy(rows.at[pl.ds(_ECH * j, _ECH)],
                            agg.at[didx.at[j]], add=True)
        plsc.subcore_barrier()
        pltpu.sync_copy(agg.at[pl.ds(s * _RPW, _RPW)],
                        out_hbm.at[c].at[pl.ds(s * _RPW, _RPW)])

    return k


def _pool_body(h_ref, s_ref, q_ref, g_ref, be_ref, out_ref):
    m = s_ref[...] / float(_N)
    v = q_ref[...] / float(_N) - m * m
    sc = lax.rsqrt(v + 1e-5) * g_ref[...]
    sh = be_ref[...] - m * sc
    hn = h_ref[...] * sc + sh
    out_ref[...] = jnp.max(hn.reshape(_RB // _NEL, _NEL, 64), axis=1)


def _maxpool(h, s, q, g, be):
    gspec = pl.BlockSpec((1, 64), lambda i: (0, 0))
    return pl.pallas_call(
        _pool_body,
        grid=(_G8,),
        in_specs=[pl.BlockSpec((_RB, 64), lambda i: (i, 0)),
                  gspec, gspec, gspec, gspec],
        out_specs=pl.BlockSpec((_RB // _NEL, 64), lambda i: (i, 0)),
        out_shape=jax.ShapeDtypeStruct((_B, 64), jnp.float32),
    )(h, s, q, g.reshape(1, 64), be.reshape(1, 64))


def _heads_body(p_ref, w5_ref, b5_ref, wc_ref, bc_ref, out_ref):
    feat = jnp.dot(p_ref[...], w5_ref[...], preferred_element_type=jnp.float32)
    feat = jnp.maximum(feat + b5_ref[...], 0.0)
    z = jnp.dot(feat, wc_ref[...], preferred_element_type=jnp.float32) + bc_ref[...]
    ps = []
    for k in range(3):
        zp = z[:, 2 * k:2 * k + 2]
        m = jnp.max(zp, axis=1, keepdims=True)
        e = jnp.exp(zp - m)
        ps.append(e / jnp.sum(e, axis=1, keepdims=True))
    p0, p1, p2 = ps
    p_hc = p0[:, 0:1] * p1[:, 0:1]
    p_ad = p0[:, 1:2] * p2[:, 1:2]
    p_ftd = p0[:, 0:1] * p1[:, 1:2] + p0[:, 1:2] * p2[:, 0:1]
    out_ref[...] = jnp.log(jnp.concatenate([p_hc, p_ftd, p_ad], axis=1) + 1e-8)


def _heads(pooled, w5, b5, wcat, bcat):
    return pl.pallas_call(
        _heads_body,
        out_shape=jax.ShapeDtypeStruct((_B, 3), jnp.float32),
    )(pooled, w5, b5, wcat, bcat)


def kernel(x, edge_index, batch, W2, b2, g3, be3, W3, b3, g4, be4, W4, b4,
           g5, be5, ew1, Wrel1, brel1, Wroot1, g6, be6, ew2, Wrel2, brel2,
           Wroot2, g7, be7, W5, b5, Whr, bhr, Whf, bhf, Wfa, bfa):
    w2pad = jnp.concatenate(
        [W2, jnp.zeros(((_NCHUNK + 1) * 128 - _NPOOL, 512), jnp.float32)], axis=0)
    h1, rs1, rq1 = _stage1(jnp.swapaxes(x, 0, 1), w2pad, b2.reshape(1, 512))
    sc1, sh1, w1t, w2t = _mk_bn_resolve((_N // 128, 128), _B * 512.0, True)(
        rs1, rq1, g3, be3, ew1.reshape(1, 60), ew2.reshape(1, 60))

    h2, rs2, rq2 = _mk_mlp_bn(512, 256)(h1, sc1, sh1, W3, b3.reshape(1, 256))
    sc2, sh2 = _mk_bn_resolve((_G8, _RB), _B * 256.0, False)(rs2, rq2, g4, be4)
    h3, rs3, rq3 = _mk_mlp_bn(256, 128)(h2, sc2, sh2, W4, b4.reshape(1, 128))
    sc3, sh3 = _mk_bn_resolve((_G8, _RB), _B * 128.0, False)(rs3, rq3, g5, be5)

    zg = jnp.zeros((1, 128), jnp.float32)
    h3n, root1 = _mk_apply_root(128, 64, True)(h3, sc3, sh3, zg, zg, Wroot1)

    src = edge_index[0]
    dst = edge_index[1]
    w1e = w1t.reshape(_NEDGE_TOT)
    w2e = w2t.reshape(_NEDGE_TOT)

    z128 = jnp.zeros((_N, 128), jnp.float32)
    parts1 = _mk_gconv_sc(128)(h3n, src, dst, w1e, z128)
    h4, s4, q4 = _mk_combine(128, 64)(parts1, root1, Wrel1, brel1.reshape(1, 64))
    h4n, root2 = _mk_apply_root(64, 64, False, dpad=128)(
        h4, s4, q4, g6.reshape(1, 64), be6.reshape(1, 64), Wroot2)

    parts2 = _mk_gconv_sc(128)(h4n, src, dst, w2e, z128)
    wrel2p = jnp.concatenate([Wrel2, jnp.zeros((64, 64), jnp.float32)], axis=0)
    h5, s5, q5 = _mk_combine(128, 64)(parts2, root2, wrel2p, brel2.reshape(1, 64))
    pooled = _maxpool(h5, s5, q5, g7, be7)
    wcat = jnp.concatenate([Whr, Whf, Wfa], axis=1)
    bcat = jnp.concatenate([bhr, bhf, bfa]).reshape(1, 6)
    return _heads(pooled, W5, b5.reshape(1, 32), wcat, bcat)


# speedup vs baseline: 3.2634x; 1.0039x over previous
import functools

import jax
import jax.numpy as jnp
from jax import lax
from jax.experimental import pallas as pl
from jax.experimental.pallas import tpu as pltpu
from jax.experimental.pallas import tpu_sc as plsc

_B = 256
_NEL = 19
_N = _B * _NEL
_D0 = 20000
_WLEN = 25
_NPOOL = 800
_CHUNK = 3200
_NCHUNK = 6
_RB = 608
_G8 = _N // _RB


def _pool_mats():
    j = jnp.arange(_CHUNK)
    pc = (j[:, None] // _WLEN == jnp.arange(128)[None, :]).astype(jnp.float32) / _WLEN
    jt = jnp.arange(_D0 - _NCHUNK * _CHUNK)
    pt = (jt[:, None] // _WLEN == jnp.arange(128)[None, :]).astype(jnp.float32) / _WLEN
    return pc, pt


def _onehot_el():
    return (jnp.arange(_N)[:, None] % _NEL == jnp.arange(_NEL)[None, :]
            ).astype(jnp.float32)


_KW = 256


def _k1a_body(xt_ref, pc_ref, pt_ref, w2_ref, b2_ref,
              h1_ref, rs_ref, rq_ref, pooled_ref):
    dn = (((0,), (0,)), ((), ()))
    for c in range(_NCHUNK):
        pooled_ref[:, 128 * c:128 * (c + 1)] = lax.dot_general(
            xt_ref[pl.ds(_CHUNK * c, _CHUNK), :], pc_ref[...], dn,
            preferred_element_type=jnp.float32)
    pooled_ref[:, _NCHUNK * 128:(_NCHUNK + 1) * 128] = lax.dot_general(
        xt_ref[pl.ds(_NCHUNK * _CHUNK, _D0 - _NCHUNK * _CHUNK), :], pt_ref[...],
        dn, preferred_element_type=jnp.float32)
    h = jnp.dot(pooled_ref[...], w2_ref[...], preferred_element_type=jnp.float32)
    h = jnp.maximum(h + b2_ref[...], 0.0)
    h1_ref[...] = h
    rs_ref[...] = jnp.sum(h, axis=1, keepdims=True)
    rq_ref[...] = jnp.sum(h * h, axis=1, keepdims=True)


def _stage1(xt, w2pad, b2):
    pc, pt = _pool_mats()
    nblk = _N // _KW
    return pl.pallas_call(
        _k1a_body,
        grid=(nblk,),
        in_specs=[
            pl.BlockSpec((_D0, _KW), lambda i: (0, i)),
            pl.BlockSpec((_CHUNK, 128), lambda i: (0, 0)),
            pl.BlockSpec((_D0 - _NCHUNK * _CHUNK, 128), lambda i: (0, 0)),
            pl.BlockSpec(((_NCHUNK + 1) * 128, 512), lambda i: (0, 0)),
            pl.BlockSpec((1, 512), lambda i: (0, 0)),
        ],
        out_specs=[
            pl.BlockSpec((_KW, 512), lambda i: (i, 0)),
            pl.BlockSpec((_KW, 1), lambda i: (i, 0)),
            pl.BlockSpec((_KW, 1), lambda i: (i, 0)),
        ],
        out_shape=[
            jax.ShapeDtypeStruct((_N, 512), jnp.float32),
            jax.ShapeDtypeStruct((_N, 1), jnp.float32),
            jax.ShapeDtypeStruct((_N, 1), jnp.float32),
        ],
        scratch_shapes=[pltpu.VMEM((_KW, (_NCHUNK + 1) * 128), jnp.float32)],
    )(xt, pc, pt, w2pad, b2)


def _mk_bn_resolve(rshape, count, with_wexp):
    rows, cols = rshape

    def body(*refs):
        if with_wexp:
            (rs_ref, rq_ref, oh_ref, g_ref, be_ref, ew1_ref, ew2_ref,
             sc_ref, sh_ref, w1t_ref, w2t_ref) = refs
            for ew_ref, wt_ref in ((ew1_ref, w1t_ref), (ew2_ref, w2t_ref)):
                ew = ew_ref[...]
                sp = jnp.maximum(ew, 0.0) + jnp.log1p(jnp.exp(-jnp.abs(ew)))
                wt_ref[...] = jnp.broadcast_to(sp, (_B, 60))
        else:
            rs_ref, rq_ref, oh_ref, g_ref, be_ref, sc_ref, sh_ref = refs
        oh = oh_ref[...]
        dn0 = (((0,), (0,)), ((), ()))
        s19 = lax.dot_general(rs_ref[...], oh, dn0,
                              preferred_element_type=jnp.float32)
        q19 = lax.dot_general(rq_ref[...], oh, dn0,
                              preferred_element_type=jnp.float32)
        m = s19 / count
        v = q19 / count - m * m
        inv = lax.rsqrt(v + 1e-5) * g_ref[...]
        sh = be_ref[...] - m * inv
        dn = (((1,), (1,)), ((), ()))
        sc_ref[...] = lax.dot_general(oh, inv, dn,
                                      preferred_element_type=jnp.float32)
        sh_ref[...] = lax.dot_general(oh, sh, dn,
                                      preferred_element_type=jnp.float32)

    out_shape = [
        jax.ShapeDtypeStruct((_N, 1), jnp.float32),
        jax.ShapeDtypeStruct((_N, 1), jnp.float32),
    ]
    if with_wexp:
        out_shape += [jax.ShapeDtypeStruct((_B, 60), jnp.float32)] * 2

    def call(rs, rq, g, be, *ews):
        return pl.pallas_call(body, out_shape=out_shape)(
            rs, rq, _onehot_el(), g.reshape(1, _NEL), be.reshape(1, _NEL), *ews)

    return call


def _mk_mlp_bn(din, dout):
    def body(h_ref, sc_ref, sh_ref, w_ref, b_ref, out_ref, rs_ref, rq_ref):
        hb = h_ref[...] * sc_ref[...] + sh_ref[...]
        h2 = jnp.dot(hb, w_ref[...], preferred_element_type=jnp.float32)
        h2 = jnp.maximum(h2 + b_ref[...], 0.0)
        out_ref[...] = h2
        rs_ref[...] = jnp.sum(h2, axis=1, keepdims=True)
        rq_ref[...] = jnp.sum(h2 * h2, axis=1, keepdims=True)

    def call(h, sc_col, sh_col, w, b):
        return pl.pallas_call(
            body,
            grid=(_G8,),
            in_specs=[
                pl.BlockSpec((_RB, din), lambda i: (i, 0)),
                pl.BlockSpec((_RB, 1), lambda i: (i, 0)),
                pl.BlockSpec((_RB, 1), lambda i: (i, 0)),
                pl.BlockSpec((din, dout), lambda i: (0, 0)),
                pl.BlockSpec((1, dout), lambda i: (0, 0)),
            ],
            out_specs=[
                pl.BlockSpec((_RB, dout), lambda i: (i, 0)),
                pl.BlockSpec((_RB, 1), lambda i: (i, 0)),
                pl.BlockSpec((_RB, 1), lambda i: (i, 0)),
            ],
            out_shape=[
                jax.ShapeDtypeStruct((_N, dout), jnp.float32),
                jax.ShapeDtypeStruct((_N, 1), jnp.float32),
                jax.ShapeDtypeStruct((_N, 1), jnp.float32),
            ],
        )(h, sc_col, sh_col, w, b)

    return call


def _mk_apply_root(din, dmid, col_scale, dpad=None):
    dpad = dpad or din

    def body(h_ref, a_ref, b_ref, g_ref, be_ref, wr_ref, hn_ref, root_ref):
        if col_scale:
            sc, sh = a_ref[...], b_ref[...]
        else:
            m = a_ref[...] / float(_N)
            v = b_ref[...] / float(_N) - m * m
            sc = lax.rsqrt(v + 1e-5) * g_ref[...]
            sh = be_ref[...] - m * sc
        hn = h_ref[...] * sc + sh
        if dpad > din:
            hn_ref[:, :din] = hn
            hn_ref[:, din:] = jnp.zeros((_RB, dpad - din), jnp.float32)
        else:
            hn_ref[...] = hn
        root_ref[...] = jnp.dot(hn, wr_ref[...], preferred_element_type=jnp.float32)

    sspec = (pl.BlockSpec((_RB, 1), lambda i: (i, 0)) if col_scale
             else pl.BlockSpec((1, din), lambda i: (0, 0)))
    gspec = pl.BlockSpec((1, din), lambda i: (0, 0))

    def call(h, a, b, g, be, wroot):
        return pl.pallas_call(
            body,
            grid=(_G8,),
            in_specs=[
                pl.BlockSpec((_RB, din), lambda i: (i, 0)),
                sspec, sspec, gspec, gspec,
                pl.BlockSpec((din, dmid), lambda i: (0, 0)),
            ],
            out_specs=[
                pl.BlockSpec((_RB, dpad), lambda i: (i, 0)),
                pl.BlockSpec((_RB, dmid), lambda i: (i, 0)),
            ],
            out_shape=[
                jax.ShapeDtypeStruct((_N, dpad), jnp.float32),
                jax.ShapeDtypeStruct((_N, dmid), jnp.float32),
            ],
        )(h, a, b, g, be, wroot)

    return call


def _mk_combine(din, dout):
    def body(p_ref, root_ref, wrel_ref, brel_ref, out_ref, s_ref, q_ref):
        agg = p_ref[0] + p_ref[1]
        h = jnp.dot(agg, wrel_ref[...], preferred_element_type=jnp.float32)
        h = jnp.maximum(h + brel_ref[...] + root_ref[...], 0.0)
        out_ref[...] = h
        ps = jnp.sum(h, axis=0).reshape(1, dout)
        pq = jnp.sum(h * h, axis=0).reshape(1, dout)

        @pl.when(pl.program_id(0) == 0)
        def _init():
            s_ref[...] = jnp.zeros_like(s_ref)
            q_ref[...] = jnp.zeros_like(q_ref)

        s_ref[...] += ps
        q_ref[...] += pq

    def call(parts, root, wrel, brel):
        return pl.pallas_call(
            body,
            grid=(_G8,),
            in_specs=[
                pl.BlockSpec((2, _RB, din), lambda i: (0, i, 0)),
                pl.BlockSpec((_RB, dout), lambda i: (i, 0)),
                pl.BlockSpec((din, dout), lambda i: (0, 0)),
                pl.BlockSpec((1, dout), lambda i: (0, 0)),
            ],
            out_specs=[
                pl.BlockSpec((_RB, dout), lambda i: (i, 0)),
                pl.BlockSpec((1, dout), lambda i: (0, 0)),
                pl.BlockSpec((1, dout), lambda i: (0, 0)),
            ],
            out_shape=[
                jax.ShapeDtypeStruct((_N, dout), jnp.float32),
                jax.ShapeDtypeStruct((1, dout), jnp.float32),
                jax.ShapeDtypeStruct((1, dout), jnp.float32),
            ],
        )(parts, root, wrel, brel)

    return call


_NEDGE_TOT = 15360
_NWORK = 32
_EPW = _NEDGE_TOT // _NWORK
_ECH = 96
_NCH = _EPW // _ECH
_RPW = _N // 16


def _mk_gconv_sc(d):
    mesh = plsc.VectorSubcoreMesh(core_axis_name="c", subcore_axis_name="s")

    @functools.partial(
        pl.kernel,
        out_type=jax.ShapeDtypeStruct((2, _N, d), jnp.float32),
        mesh=mesh,
        scratch_types=[
            pltpu.VMEM((_NCH, _ECH), jnp.int32),
            pltpu.VMEM((_NCH, _ECH), jnp.int32),
            pltpu.VMEM((_EPW,), jnp.float32),
            pltpu.VMEM((_EPW, d), jnp.float32),
            pltpu.VMEM_SHARED((_N, d), jnp.float32),
            pltpu.SemaphoreType.DMA,
        ],
    )
    def k(h_hbm, src_hbm, dst_hbm, w_hbm, z_hbm, out_hbm,
          sidx, didx, wv, rows, agg, sem):
        c = lax.axis_index("c")
        s = lax.axis_index("s")
        wid = s * 2 + c
        base = wid * _EPW
        pltpu.sync_copy(z_hbm.at[pl.ds(s * _RPW, _RPW)],
                        agg.at[pl.ds(s * _RPW, _RPW)])
        for j in range(_NCH):
            pltpu.sync_copy(src_hbm.at[pl.ds(base + _ECH * j, _ECH)], sidx.at[j])
            pltpu.sync_copy(dst_hbm.at[pl.ds(base + _ECH * j, _ECH)], didx.at[j])
        pltpu.sync_copy(w_hbm.at[pl.ds(base, _EPW)], wv)
        for j in range(_NCH):
            pltpu.async_copy(h_hbm.at[sidx.at[j]],
                             rows.at[pl.ds(_ECH * j, _ECH)], sem).wait()

        def scale(g, carry):
            wg = wv[pl.ds(g * 16, 16)]
            for i in range(16):
                e = g * 16 + i
                wvec = jnp.full((16,), 0.0, jnp.float32) + wg[i]
                for q in range(d // 16):
                    sl = pl.ds(q * 16, 16)
                    rows[e, sl] = rows[e, sl] * wvec
            return carry

        lax.fori_loop(0, _EPW // 16, scale, 0)
        plsc.subcore_barrier()
        for j in range(_NCH):
            pltpu.sync_copy(rows.at[pl.ds(_ECH * j, _ECH)],
                            agg.at[didx.at[j]], add=True)
        plsc.subcore_barrier()
        pltpu.sync_copy(agg.at[pl.ds(s * _RPW, _RPW)],
                        out_hbm.at[c].at[pl.ds(s * _RPW, _RPW)])

    return k


def _pool_body(h_ref, s_ref, q_ref, g_ref, be_ref, out_ref):
    m = s_ref[...] / float(_N)
    v = q_ref[...] / float(_N) - m * m
    sc = lax.rsqrt(v + 1e-5) * g_ref[...]
    sh = be_ref[...] - m * sc
    hn = h_ref[...] * sc + sh
    out_ref[...] = jnp.max(hn.reshape(_RB // _NEL, _NEL, 64), axis=1)


def _maxpool(h, s, q, g, be):
    gspec = pl.BlockSpec((1, 64), lambda i: (0, 0))
    return pl.pallas_call(
        _pool_body,
        grid=(_G8,),
        in_specs=[pl.BlockSpec((_RB, 64), lambda i: (i, 0)),
                  gspec, gspec, gspec, gspec],
        out_specs=pl.BlockSpec((_RB // _NEL, 64), lambda i: (i, 0)),
        out_shape=jax.ShapeDtypeStruct((_B, 64), jnp.float32),
    )(h, s, q, g.reshape(1, 64), be.reshape(1, 64))


def _heads_body(p_ref, w5_ref, b5_ref, wc_ref, bc_ref, out_ref):
    feat = jnp.dot(p_ref[...], w5_ref[...], preferred_element_type=jnp.float32)
    feat = jnp.maximum(feat + b5_ref[...], 0.0)
    z = jnp.dot(feat, wc_ref[...], preferred_element_type=jnp.float32) + bc_ref[...]
    ps = []
    for k in range(3):
        zp = z[:, 2 * k:2 * k + 2]
        m = jnp.max(zp, axis=1, keepdims=True)
        e = jnp.exp(zp - m)
        ps.append(e / jnp.sum(e, axis=1, keepdims=True))
    p0, p1, p2 = ps
    p_hc = p0[:, 0:1] * p1[:, 0:1]
    p_ad = p0[:, 1:2] * p2[:, 1:2]
    p_ftd = p0[:, 0:1] * p1[:, 1:2] + p0[:, 1:2] * p2[:, 0:1]
    out_ref[...] = jnp.log(jnp.concatenate([p_hc, p_ftd, p_ad], axis=1) + 1e-8)


def _heads(pooled, w5, b5, wcat, bcat):
    return pl.pallas_call(
        _heads_body,
        out_shape=jax.ShapeDtypeStruct((_B, 3), jnp.float32),
    )(pooled, w5, b5, wcat, bcat)


def kernel(x, edge_index, batch, W2, b2, g3, be3, W3, b3, g4, be4, W4, b4,
           g5, be5, ew1, Wrel1, brel1, Wroot1, g6, be6, ew2, Wrel2, brel2,
           Wroot2, g7, be7, W5, b5, Whr, bhr, Whf, bhf, Wfa, bfa):
    w2pad = jnp.concatenate(
        [W2, jnp.zeros(((_NCHUNK + 1) * 128 - _NPOOL, 512), jnp.float32)], axis=0)
    h1, rs1, rq1 = _stage1(jnp.swapaxes(x, 0, 1), w2pad, b2.reshape(1, 512))
    sc1, sh1, w1t, w2t = _mk_bn_resolve((_N // 128, 128), _B * 512.0, True)(
        rs1, rq1, g3, be3, ew1.reshape(1, 60), ew2.reshape(1, 60))

    h2, rs2, rq2 = _mk_mlp_bn(512, 256)(h1, sc1, sh1, W3, b3.reshape(1, 256))
    sc2, sh2 = _mk_bn_resolve((_G8, _RB), _B * 256.0, False)(rs2, rq2, g4, be4)
    h3, rs3, rq3 = _mk_mlp_bn(256, 128)(h2, sc2, sh2, W4, b4.reshape(1, 128))
    sc3, sh3 = _mk_bn_resolve((_G8, _RB), _B * 128.0, False)(rs3, rq3, g5, be5)

    zg = jnp.zeros((1, 128), jnp.float32)
    h3n, root1 = _mk_apply_root(128, 64, True)(h3, sc3, sh3, zg, zg, Wroot1)

    src = edge_index[0]
    dst = edge_index[1]
    w1e = w1t.reshape(_NEDGE_TOT)
    w2e = w2t.reshape(_NEDGE_TOT)

    z128 = jnp.zeros((_N, 128), jnp.float32)
    parts1 = _mk_gconv_sc(128)(h3n, src, dst, w1e, z128)
    h4, s4, q4 = _mk_combine(128, 64)(parts1, root1, Wrel1, brel1.reshape(1, 64))
    h4n, root2 = _mk_apply_root(64, 64, False, dpad=128)(
        h4, s4, q4, g6.reshape(1, 64), be6.reshape(1, 64), Wroot2)

    parts2 = _mk_gconv_sc(128)(h4n, src, dst, w2e, z128)
    wrel2p = jnp.concatenate([Wrel2, jnp.zeros((64, 64), jnp.float32)], axis=0)
    h5, s5, q5 = _mk_combine(128, 64)(parts2, root2, wrel2p, brel2.reshape(1, 64))
    pooled = _maxpool(h5, s5, q5, g7, be7)
    wcat = jnp.concatenate([Whr, Whf, Wfa], axis=1)
    bcat = jnp.concatenate([bhr, bhf, bfa]).reshape(1, 6)
    return _heads(pooled, W5, b5.reshape(1, 32), wcat, bcat)


# speedup vs baseline: 3.4255x; 1.0497x over previous
import functools

import jax
import jax.numpy as jnp
from jax import lax
from jax.experimental import pallas as pl
from jax.experimental.pallas import tpu as pltpu
from jax.experimental.pallas import tpu_sc as plsc

_B = 256
_NEL = 19
_N = _B * _NEL
_D0 = 20000
_WLEN = 25
_NPOOL = 800
_CHUNK = 3200
_NCHUNK = 6
_RB = 608
_G8 = _N // _RB


def _pool_mats():
    j = jnp.arange(_CHUNK)
    pc = (j[:, None] // _WLEN == jnp.arange(128)[None, :]).astype(jnp.float32) / _WLEN
    jt = jnp.arange(_D0 - _NCHUNK * _CHUNK)
    pt = (jt[:, None] // _WLEN == jnp.arange(128)[None, :]).astype(jnp.float32) / _WLEN
    return pc, pt


def _onehot_el():
    return (jnp.arange(_N)[:, None] % _NEL == jnp.arange(_NEL)[None, :]
            ).astype(jnp.float32)


_KW = 256


def _k1a_body(xt_ref, pc_ref, pt_ref, w2_ref, b2_ref,
              h1_ref, rs_ref, rq_ref, pooled_ref):
    dn = (((0,), (0,)), ((), ()))
    for c in range(_NCHUNK):
        pooled_ref[:, 128 * c:128 * (c + 1)] = lax.dot_general(
            xt_ref[pl.ds(_CHUNK * c, _CHUNK), :], pc_ref[...], dn,
            preferred_element_type=jnp.float32)
    pooled_ref[:, _NCHUNK * 128:(_NCHUNK + 1) * 128] = lax.dot_general(
        xt_ref[pl.ds(_NCHUNK * _CHUNK, _D0 - _NCHUNK * _CHUNK), :], pt_ref[...],
        dn, preferred_element_type=jnp.float32)
    h = jnp.dot(pooled_ref[...], w2_ref[...], preferred_element_type=jnp.float32)
    h = jnp.maximum(h + b2_ref[...], 0.0)
    h1_ref[...] = h
    rs_ref[...] = jnp.sum(h, axis=1, keepdims=True)
    rq_ref[...] = jnp.sum(h * h, axis=1, keepdims=True)


def _stage1(xt, w2pad, b2):
    pc, pt = _pool_mats()
    nblk = _N // _KW
    return pl.pallas_call(
        _k1a_body,
        grid=(nblk,),
        in_specs=[
            pl.BlockSpec((_D0, _KW), lambda i: (0, i)),
            pl.BlockSpec((_CHUNK, 128), lambda i: (0, 0)),
            pl.BlockSpec((_D0 - _NCHUNK * _CHUNK, 128), lambda i: (0, 0)),
            pl.BlockSpec(((_NCHUNK + 1) * 128, 512), lambda i: (0, 0)),
            pl.BlockSpec((1, 512), lambda i: (0, 0)),
        ],
        out_specs=[
            pl.BlockSpec((_KW, 512), lambda i: (i, 0)),
            pl.BlockSpec((_KW, 1), lambda i: (i, 0)),
            pl.BlockSpec((_KW, 1), lambda i: (i, 0)),
        ],
        out_shape=[
            jax.ShapeDtypeStruct((_N, 512), jnp.float32),
            jax.ShapeDtypeStruct((_N, 1), jnp.float32),
            jax.ShapeDtypeStruct((_N, 1), jnp.float32),
        ],
        scratch_shapes=[pltpu.VMEM((_KW, (_NCHUNK + 1) * 128), jnp.float32)],
    )(xt, pc, pt, w2pad, b2)


def _mk_bn_resolve(rshape, count, with_wexp):
    rows, cols = rshape

    def body(*refs):
        if with_wexp:
            (rs_ref, rq_ref, oh_ref, g_ref, be_ref, ew1_ref, ew2_ref,
             sc_ref, sh_ref, w1t_ref, w2t_ref) = refs
            for ew_ref, wt_ref in ((ew1_ref, w1t_ref), (ew2_ref, w2t_ref)):
                ew = ew_ref[...]
                sp = jnp.maximum(ew, 0.0) + jnp.log1p(jnp.exp(-jnp.abs(ew)))
                wt_ref[...] = jnp.broadcast_to(sp, (_B, 60))
        else:
            rs_ref, rq_ref, oh_ref, g_ref, be_ref, sc_ref, sh_ref = refs
        oh = oh_ref[...]
        dn0 = (((0,), (0,)), ((), ()))
        s19 = lax.dot_general(rs_ref[...], oh, dn0,
                              preferred_element_type=jnp.float32)
        q19 = lax.dot_general(rq_ref[...], oh, dn0,
                              preferred_element_type=jnp.float32)
        m = s19 / count
        v = q19 / count - m * m
        inv = lax.rsqrt(v + 1e-5) * g_ref[...]
        sh = be_ref[...] - m * inv
        dn = (((1,), (1,)), ((), ()))
        sc_ref[...] = lax.dot_general(oh, inv, dn,
                                      preferred_element_type=jnp.float32)
        sh_ref[...] = lax.dot_general(oh, sh, dn,
                                      preferred_element_type=jnp.float32)

    out_shape = [
        jax.ShapeDtypeStruct((_N, 1), jnp.float32),
        jax.ShapeDtypeStruct((_N, 1), jnp.float32),
    ]
    if with_wexp:
        out_shape += [jax.ShapeDtypeStruct((_B, 60), jnp.float32)] * 2

    def call(rs, rq, g, be, *ews):
        return pl.pallas_call(body, out_shape=out_shape)(
            rs, rq, _onehot_el(), g.reshape(1, _NEL), be.reshape(1, _NEL), *ews)

    return call


def _mk_mlp_bn(din, dout):
    def body(h_ref, sc_ref, sh_ref, w_ref, b_ref, out_ref, rs_ref, rq_ref):
        hb = h_ref[...] * sc_ref[...] + sh_ref[...]
        h2 = jnp.dot(hb, w_ref[...], preferred_element_type=jnp.float32)
        h2 = jnp.maximum(h2 + b_ref[...], 0.0)
        out_ref[...] = h2
        rs_ref[...] = jnp.sum(h2, axis=1, keepdims=True)
        rq_ref[...] = jnp.sum(h2 * h2, axis=1, keepdims=True)

    def call(h, sc_col, sh_col, w, b):
        return pl.pallas_call(
            body,
            grid=(_G8,),
            in_specs=[
                pl.BlockSpec((_RB, din), lambda i: (i, 0)),
                pl.BlockSpec((_RB, 1), lambda i: (i, 0)),
                pl.BlockSpec((_RB, 1), lambda i: (i, 0)),
                pl.BlockSpec((din, dout), lambda i: (0, 0)),
                pl.BlockSpec((1, dout), lambda i: (0, 0)),
            ],
            out_specs=[
                pl.BlockSpec((_RB, dout), lambda i: (i, 0)),
                pl.BlockSpec((_RB, 1), lambda i: (i, 0)),
                pl.BlockSpec((_RB, 1), lambda i: (i, 0)),
            ],
            out_shape=[
                jax.ShapeDtypeStruct((_N, dout), jnp.float32),
                jax.ShapeDtypeStruct((_N, 1), jnp.float32),
                jax.ShapeDtypeStruct((_N, 1), jnp.float32),
            ],
        )(h, sc_col, sh_col, w, b)

    return call


def _mk_apply_root(din, dmid, col_scale, dpad=None):
    dpad = dpad or din

    def body(h_ref, a_ref, b_ref, g_ref, be_ref, wr_ref, hn_ref, root_ref):
        if col_scale:
            sc, sh = a_ref[...], b_ref[...]
        else:
            m = a_ref[...] / float(_N)
            v = b_ref[...] / float(_N) - m * m
            sc = lax.rsqrt(v + 1e-5) * g_ref[...]
            sh = be_ref[...] - m * sc
        hn = h_ref[...] * sc + sh
        if dpad > din:
            hn_ref[:, :din] = hn
            hn_ref[:, din:] = jnp.zeros((_RB, dpad - din), jnp.float32)
        else:
            hn_ref[...] = hn
        root_ref[...] = jnp.dot(hn, wr_ref[...], preferred_element_type=jnp.float32)

    sspec = (pl.BlockSpec((_RB, 1), lambda i: (i, 0)) if col_scale
             else pl.BlockSpec((1, din), lambda i: (0, 0)))
    gspec = pl.BlockSpec((1, din), lambda i: (0, 0))

    def call(h, a, b, g, be, wroot):
        return pl.pallas_call(
            body,
            grid=(_G8,),
            in_specs=[
                pl.BlockSpec((_RB, din), lambda i: (i, 0)),
                sspec, sspec, gspec, gspec,
                pl.BlockSpec((din, dmid), lambda i: (0, 0)),
            ],
            out_specs=[
                pl.BlockSpec((_RB, dpad), lambda i: (i, 0)),
                pl.BlockSpec((_RB, dmid), lambda i: (i, 0)),
            ],
            out_shape=[
                jax.ShapeDtypeStruct((_N, dpad), jnp.float32),
                jax.ShapeDtypeStruct((_N, dmid), jnp.float32),
            ],
        )(h, a, b, g, be, wroot)

    return call


def _mk_combine(din, dout):
    def body(p_ref, root_ref, wrel_ref, brel_ref, out_ref, s_ref, q_ref):
        agg = p_ref[0] + p_ref[1]
        h = jnp.dot(agg, wrel_ref[...], preferred_element_type=jnp.float32)
        h = jnp.maximum(h + brel_ref[...] + root_ref[...], 0.0)
        out_ref[...] = h
        ps = jnp.sum(h, axis=0).reshape(1, dout)
        pq = jnp.sum(h * h, axis=0).reshape(1, dout)

        @pl.when(pl.program_id(0) == 0)
        def _init():
            s_ref[...] = jnp.zeros_like(s_ref)
            q_ref[...] = jnp.zeros_like(q_ref)

        s_ref[...] += ps
        q_ref[...] += pq

    def call(parts, root, wrel, brel):
        return pl.pallas_call(
            body,
            grid=(_G8,),
            in_specs=[
                pl.BlockSpec((2, _RB, din), lambda i: (0, i, 0)),
                pl.BlockSpec((_RB, dout), lambda i: (i, 0)),
                pl.BlockSpec((din, dout), lambda i: (0, 0)),
                pl.BlockSpec((1, dout), lambda i: (0, 0)),
            ],
            out_specs=[
                pl.BlockSpec((_RB, dout), lambda i: (i, 0)),
                pl.BlockSpec((1, dout), lambda i: (0, 0)),
                pl.BlockSpec((1, dout), lambda i: (0, 0)),
            ],
            out_shape=[
                jax.ShapeDtypeStruct((_N, dout), jnp.float32),
                jax.ShapeDtypeStruct((1, dout), jnp.float32),
                jax.ShapeDtypeStruct((1, dout), jnp.float32),
            ],
        )(parts, root, wrel, brel)

    return call


_NEDGE_TOT = 15360
_NWORK = 32
_EPW = _NEDGE_TOT // _NWORK
_ECH = 96
_NCH = _EPW // _ECH
_RPW = _N // 16


def _mk_gconv_sc(d):
    mesh = plsc.VectorSubcoreMesh(core_axis_name="c", subcore_axis_name="s")

    @functools.partial(
        pl.kernel,
        out_type=jax.ShapeDtypeStruct((2, _N, d), jnp.float32),
        mesh=mesh,
        scratch_types=[
            pltpu.VMEM((_EPW,), jnp.int32),
            pltpu.VMEM((_NCH, _ECH), jnp.int32),
            pltpu.VMEM((_EPW,), jnp.float32),
            pltpu.VMEM((_EPW, d), jnp.float32),
            pltpu.VMEM_SHARED((_N, d), jnp.float32),
            pltpu.SemaphoreType.DMA,
            pltpu.SemaphoreType.DMA,
        ],
    )
    def k(h_hbm, src_hbm, dst_hbm, w_hbm, z_hbm, out_hbm,
          sidx, didx, wv, rows, agg, sem, zsem):
        c = lax.axis_index("c")
        s = lax.axis_index("s")
        wid = s * 2 + c
        base = wid * _EPW
        zcp = pltpu.async_copy(z_hbm.at[pl.ds(s * _RPW, _RPW)],
                               agg.at[pl.ds(s * _RPW, _RPW)], zsem)
        pltpu.sync_copy(src_hbm.at[pl.ds(base, _EPW)], sidx)
        for j in range(_NCH):
            pltpu.sync_copy(dst_hbm.at[pl.ds(base + _ECH * j, _ECH)], didx.at[j])
        pltpu.sync_copy(w_hbm.at[pl.ds(base, _EPW)], wv)
        cps = [pltpu.async_copy(h_hbm.at[sidx.at[pl.ds(_ECH * j, _ECH)]],
                                rows.at[pl.ds(_ECH * j, _ECH)], sem)
               for j in range(_NCH)]
        for cp in cps:
            cp.wait()

        def scale(g, carry):
            wg = wv[pl.ds(g * 16, 16)]
            for i in range(16):
                e = g * 16 + i
                wvec = jnp.full((16,), 0.0, jnp.float32) + wg[i]
                for q in range(d // 16):
                    sl = pl.ds(q * 16, 16)
                    rows[e, sl] = rows[e, sl] * wvec
            return carry

        lax.fori_loop(0, _EPW // 16, scale, 0)
        zcp.wait()
        plsc.subcore_barrier()
        scps = [pltpu.async_copy(rows.at[pl.ds(_ECH * j, _ECH)],
                                 agg.at[didx.at[j]], sem, add=True)
                for j in range(_NCH)]
        for cp in scps:
            cp.wait()
        plsc.subcore_barrier()
        pltpu.sync_copy(agg.at[pl.ds(s * _RPW, _RPW)],
                        out_hbm.at[c].at[pl.ds(s * _RPW, _RPW)])

    return k


def _pool_body(h_ref, s_ref, q_ref, g_ref, be_ref, out_ref):
    m = s_ref[...] / float(_N)
    v = q_ref[...] / float(_N) - m * m
    sc = lax.rsqrt(v + 1e-5) * g_ref[...]
    sh = be_ref[...] - m * sc
    hn = h_ref[...] * sc + sh
    out_ref[...] = jnp.max(hn.reshape(_RB // _NEL, _NEL, 64), axis=1)


def _maxpool(h, s, q, g, be):
    gspec = pl.BlockSpec((1, 64), lambda i: (0, 0))
    return pl.pallas_call(
        _pool_body,
        grid=(_G8,),
        in_specs=[pl.BlockSpec((_RB, 64), lambda i: (i, 0)),
                  gspec, gspec, gspec, gspec],
        out_specs=pl.BlockSpec((_RB // _NEL, 64), lambda i: (i, 0)),
        out_shape=jax.ShapeDtypeStruct((_B, 64), jnp.float32),
    )(h, s, q, g.reshape(1, 64), be.reshape(1, 64))


def _heads_body(p_ref, w5_ref, b5_ref, wc_ref, bc_ref, out_ref):
    feat = jnp.dot(p_ref[...], w5_ref[...], preferred_element_type=jnp.float32)
    feat = jnp.maximum(feat + b5_ref[...], 0.0)
    z = jnp.dot(feat, wc_ref[...], preferred_element_type=jnp.float32) + bc_ref[...]
    ps = []
    for k in range(3):
        zp = z[:, 2 * k:2 * k + 2]
        m = jnp.max(zp, axis=1, keepdims=True)
        e = jnp.exp(zp - m)
        ps.append(e / jnp.sum(e, axis=1, keepdims=True))
    p0, p1, p2 = ps
    p_hc = p0[:, 0:1] * p1[:, 0:1]
    p_ad = p0[:, 1:2] * p2[:, 1:2]
    p_ftd = p0[:, 0:1] * p1[:, 1:2] + p0[:, 1:2] * p2[:, 0:1]
    out_ref[...] = jnp.log(jnp.concatenate([p_hc, p_ftd, p_ad], axis=1) + 1e-8)


def _heads(pooled, w5, b5, wcat, bcat):
    return pl.pallas_call(
        _heads_body,
        out_shape=jax.ShapeDtypeStruct((_B, 3), jnp.float32),
    )(pooled, w5, b5, wcat, bcat)


def kernel(x, edge_index, batch, W2, b2, g3, be3, W3, b3, g4, be4, W4, b4,
           g5, be5, ew1, Wrel1, brel1, Wroot1, g6, be6, ew2, Wrel2, brel2,
           Wroot2, g7, be7, W5, b5, Whr, bhr, Whf, bhf, Wfa, bfa):
    w2pad = jnp.concatenate(
        [W2, jnp.zeros(((_NCHUNK + 1) * 128 - _NPOOL, 512), jnp.float32)], axis=0)
    h1, rs1, rq1 = _stage1(jnp.swapaxes(x, 0, 1), w2pad, b2.reshape(1, 512))
    sc1, sh1, w1t, w2t = _mk_bn_resolve((_N // 128, 128), _B * 512.0, True)(
        rs1, rq1, g3, be3, ew1.reshape(1, 60), ew2.reshape(1, 60))

    h2, rs2, rq2 = _mk_mlp_bn(512, 256)(h1, sc1, sh1, W3, b3.reshape(1, 256))
    sc2, sh2 = _mk_bn_resolve((_G8, _RB), _B * 256.0, False)(rs2, rq2, g4, be4)
    h3, rs3, rq3 = _mk_mlp_bn(256, 128)(h2, sc2, sh2, W4, b4.reshape(1, 128))
    sc3, sh3 = _mk_bn_resolve((_G8, _RB), _B * 128.0, False)(rs3, rq3, g5, be5)

    zg = jnp.zeros((1, 128), jnp.float32)
    h3n, root1 = _mk_apply_root(128, 64, True)(h3, sc3, sh3, zg, zg, Wroot1)

    src = edge_index[0]
    dst = edge_index[1]
    w1e = w1t.reshape(_NEDGE_TOT)
    w2e = w2t.reshape(_NEDGE_TOT)

    z128 = jnp.zeros((_N, 128), jnp.float32)
    parts1 = _mk_gconv_sc(128)(h3n, src, dst, w1e, z128)
    h4, s4, q4 = _mk_combine(128, 64)(parts1, root1, Wrel1, brel1.reshape(1, 64))
    h4n, root2 = _mk_apply_root(64, 64, False, dpad=128)(
        h4, s4, q4, g6.reshape(1, 64), be6.reshape(1, 64), Wroot2)

    parts2 = _mk_gconv_sc(128)(h4n, src, dst, w2e, z128)
    wrel2p = jnp.concatenate([Wrel2, jnp.zeros((64, 64), jnp.float32)], axis=0)
    h5, s5, q5 = _mk_combine(128, 64)(parts2, root2, wrel2p, brel2.reshape(1, 64))
    pooled = _maxpool(h5, s5, q5, g7, be7)
    wcat = jnp.concatenate([Whr, Whf, Wfa], axis=1)
    bcat = jnp.concatenate([bhr, bhf, bfa]).reshape(1, 6)
    return _heads(pooled, W5, b5.reshape(1, 32), wcat, bcat)


# speedup vs baseline: 3.5612x; 1.0396x over previous
import functools

import jax
import jax.numpy as jnp
from jax import lax
from jax.experimental import pallas as pl
from jax.experimental.pallas import tpu as pltpu
from jax.experimental.pallas import tpu_sc as plsc

_B = 256
_NEL = 19
_N = _B * _NEL
_D0 = 20000
_WLEN = 25
_NPOOL = 800
_CHUNK = 3200
_NCHUNK = 6
_RB = 608
_G8 = _N // _RB


def _pool_mats():
    j = jnp.arange(_CHUNK)
    pc = (j[:, None] // _WLEN == jnp.arange(128)[None, :]).astype(jnp.float32) / _WLEN
    jt = jnp.arange(_D0 - _NCHUNK * _CHUNK)
    pt = (jt[:, None] // _WLEN == jnp.arange(128)[None, :]).astype(jnp.float32) / _WLEN
    return pc, pt


def _onehot_el():
    return (jnp.arange(_N)[:, None] % _NEL == jnp.arange(_NEL)[None, :]
            ).astype(jnp.float32)


_KW = 256


def _k1a_body(xt_ref, pc_ref, pt_ref, w2_ref, b2_ref,
              h1_ref, rs_ref, rq_ref, pooled_ref):
    dn = (((0,), (0,)), ((), ()))
    for c in range(_NCHUNK):
        pooled_ref[:, 128 * c:128 * (c + 1)] = lax.dot_general(
            xt_ref[pl.ds(_CHUNK * c, _CHUNK), :], pc_ref[...], dn,
            preferred_element_type=jnp.float32)
    pooled_ref[:, _NCHUNK * 128:(_NCHUNK + 1) * 128] = lax.dot_general(
        xt_ref[pl.ds(_NCHUNK * _CHUNK, _D0 - _NCHUNK * _CHUNK), :], pt_ref[...],
        dn, preferred_element_type=jnp.float32)
    h = jnp.dot(pooled_ref[...], w2_ref[...], preferred_element_type=jnp.float32)
    h = jnp.maximum(h + b2_ref[...], 0.0)
    h1_ref[...] = h
    rs_ref[...] = jnp.sum(h, axis=1, keepdims=True)
    rq_ref[...] = jnp.sum(h * h, axis=1, keepdims=True)


def _stage1(xt, w2pad, b2):
    pc, pt = _pool_mats()
    nblk = _N // _KW
    return pl.pallas_call(
        _k1a_body,
        grid=(nblk,),
        in_specs=[
            pl.BlockSpec((_D0, _KW), lambda i: (0, i)),
            pl.BlockSpec((_CHUNK, 128), lambda i: (0, 0)),
            pl.BlockSpec((_D0 - _NCHUNK * _CHUNK, 128), lambda i: (0, 0)),
            pl.BlockSpec(((_NCHUNK + 1) * 128, 512), lambda i: (0, 0)),
            pl.BlockSpec((1, 512), lambda i: (0, 0)),
        ],
        out_specs=[
            pl.BlockSpec((_KW, 512), lambda i: (i, 0)),
            pl.BlockSpec((_KW, 1), lambda i: (i, 0)),
            pl.BlockSpec((_KW, 1), lambda i: (i, 0)),
        ],
        out_shape=[
            jax.ShapeDtypeStruct((_N, 512), jnp.float32),
            jax.ShapeDtypeStruct((_N, 1), jnp.float32),
            jax.ShapeDtypeStruct((_N, 1), jnp.float32),
        ],
        scratch_shapes=[pltpu.VMEM((_KW, (_NCHUNK + 1) * 128), jnp.float32)],
    )(xt, pc, pt, w2pad, b2)


def _resolve_cols(rs_ref, rq_ref, oh_ref, g_ref, be_ref, count, scc_ref, shc_ref):
    oh = oh_ref[...]
    dn0 = (((0,), (0,)), ((), ()))
    s19 = lax.dot_general(rs_ref[...], oh, dn0, preferred_element_type=jnp.float32)
    q19 = lax.dot_general(rq_ref[...], oh, dn0, preferred_element_type=jnp.float32)
    m = s19 / count
    v = q19 / count - m * m
    inv = lax.rsqrt(v + 1e-5) * g_ref[...]
    sh = be_ref[...] - m * inv
    dn1 = (((1,), (1,)), ((), ()))
    scc_ref[...] = lax.dot_general(oh, inv, dn1, preferred_element_type=jnp.float32)
    shc_ref[...] = lax.dot_general(oh, sh, dn1, preferred_element_type=jnp.float32)


def _mk_mlp_bn(din, dout, count, wexp=False):
    def body(*refs):
        if wexp:
            (h_ref, rs_ref, rq_ref, oh_ref, g_ref, be_ref, w_ref, b_ref,
             ew1_ref, ew2_ref, out_ref, rso_ref, rqo_ref, w1t_ref, w2t_ref,
             scc_ref, shc_ref) = refs
        else:
            (h_ref, rs_ref, rq_ref, oh_ref, g_ref, be_ref, w_ref, b_ref,
             out_ref, rso_ref, rqo_ref, scc_ref, shc_ref) = refs
        i = pl.program_id(0)

        @pl.when(i == 0)
        def _resolve():
            _resolve_cols(rs_ref, rq_ref, oh_ref, g_ref, be_ref, count,
                          scc_ref, shc_ref)
            if wexp:
                for ew_ref, wt_ref in ((ew1_ref, w1t_ref), (ew2_ref, w2t_ref)):
                    ew = ew_ref[...]
                    sp = jnp.maximum(ew, 0.0) + jnp.log1p(jnp.exp(-jnp.abs(ew)))
                    wt_ref[...] = jnp.broadcast_to(sp, (_B, 60))

        off = pl.multiple_of(i * _RB, _RB)
        sc = scc_ref[pl.ds(off, _RB), :]
        sh = shc_ref[pl.ds(off, _RB), :]
        hb = h_ref[...] * sc + sh
        h2 = jnp.dot(hb, w_ref[...], preferred_element_type=jnp.float32)
        h2 = jnp.maximum(h2 + b_ref[...], 0.0)
        out_ref[...] = h2
        rso_ref[...] = jnp.sum(h2, axis=1, keepdims=True)
        rqo_ref[...] = jnp.sum(h2 * h2, axis=1, keepdims=True)

    full = lambda shape: pl.BlockSpec(shape, lambda i: tuple(0 for _ in shape))
    out_shape = [
        jax.ShapeDtypeStruct((_N, dout), jnp.float32),
        jax.ShapeDtypeStruct((_N, 1), jnp.float32),
        jax.ShapeDtypeStruct((_N, 1), jnp.float32),
    ]
    out_specs = [
        pl.BlockSpec((_RB, dout), lambda i: (i, 0)),
        pl.BlockSpec((_RB, 1), lambda i: (i, 0)),
        pl.BlockSpec((_RB, 1), lambda i: (i, 0)),
    ]
    if wexp:
        out_shape += [jax.ShapeDtypeStruct((_B, 60), jnp.float32)] * 2
        out_specs += [full((_B, 60))] * 2

    def call(h, rs, rq, g, be, w, b, *ews):
        return pl.pallas_call(
            body,
            grid=(_G8,),
            in_specs=[
                pl.BlockSpec((_RB, din), lambda i: (i, 0)),
                full((_N, 1)), full((_N, 1)), full((_N, _NEL)),
                full((1, _NEL)), full((1, _NEL)),
                full((din, dout)), full((1, dout)),
            ] + ([full((1, 60))] * 2 if wexp else []),
            out_specs=out_specs,
            out_shape=out_shape,
            scratch_shapes=[pltpu.VMEM((_N, 1), jnp.float32),
                            pltpu.VMEM((_N, 1), jnp.float32)],
        )(h, rs, rq, _onehot_el(), g.reshape(1, _NEL), be.reshape(1, _NEL),
          w, b, *ews)

    return call


def _mk_apply_root(din, dmid, col_scale, dpad=None):
    dpad = dpad or din

    def body(h_ref, a_ref, b_ref, oh_ref, g_ref, be_ref, wr_ref,
             hn_ref, root_ref, scc_ref, shc_ref):
        i = pl.program_id(0)
        if col_scale:
            @pl.when(i == 0)
            def _resolve():
                _resolve_cols(a_ref, b_ref, oh_ref, g_ref, be_ref,
                              _B * float(din), scc_ref, shc_ref)

            off = pl.multiple_of(i * _RB, _RB)
            sc = scc_ref[pl.ds(off, _RB), :]
            sh = shc_ref[pl.ds(off, _RB), :]
        else:
            m = a_ref[...] / float(_N)
            v = b_ref[...] / float(_N) - m * m
            sc = lax.rsqrt(v + 1e-5) * g_ref[...]
            sh = be_ref[...] - m * sc
        hn = h_ref[...] * sc + sh
        if dpad > din:
            hn_ref[:, :din] = hn
            hn_ref[:, din:] = jnp.zeros((_RB, dpad - din), jnp.float32)
        else:
            hn_ref[...] = hn
        root_ref[...] = jnp.dot(hn, wr_ref[...], preferred_element_type=jnp.float32)

    full = lambda shape: pl.BlockSpec(shape, lambda i: tuple(0 for _ in shape))
    sspec = (full((_N, 1)) if col_scale else full((1, din)))
    gspec = (full((1, _NEL)) if col_scale else full((1, din)))

    def call(h, a, b, g, be, wroot):
        return pl.pallas_call(
            body,
            grid=(_G8,),
            in_specs=[
                pl.BlockSpec((_RB, din), lambda i: (i, 0)),
                sspec, sspec, full((_N, _NEL)), gspec, gspec,
                full((din, dmid)),
            ],
            out_specs=[
                pl.BlockSpec((_RB, dpad), lambda i: (i, 0)),
                pl.BlockSpec((_RB, dmid), lambda i: (i, 0)),
            ],
            out_shape=[
                jax.ShapeDtypeStruct((_N, dpad), jnp.float32),
                jax.ShapeDtypeStruct((_N, dmid), jnp.float32),
            ],
            scratch_shapes=[pltpu.VMEM((_N, 1), jnp.float32),
                            pltpu.VMEM((_N, 1), jnp.float32)],
        )(h, a, b, _onehot_el(),
          g.reshape(1, -1) if g.ndim == 1 else g,
          be.reshape(1, -1) if be.ndim == 1 else be, wroot)

    return call


def _mk_combine(din, dout):
    def body(p_ref, root_ref, wrel_ref, brel_ref, out_ref, s_ref, q_ref):
        agg = p_ref[0] + p_ref[1]
        h = jnp.dot(agg, wrel_ref[...], preferred_element_type=jnp.float32)
        h = jnp.maximum(h + brel_ref[...] + root_ref[...], 0.0)
        out_ref[...] = h
        ps = jnp.sum(h, axis=0).reshape(1, dout)
        pq = jnp.sum(h * h, axis=0).reshape(1, dout)

        @pl.when(pl.program_id(0) == 0)
        def _init():
            s_ref[...] = jnp.zeros_like(s_ref)
            q_ref[...] = jnp.zeros_like(q_ref)

        s_ref[...] += ps
        q_ref[...] += pq

    def call(parts, root, wrel, brel):
        return pl.pallas_call(
            body,
            grid=(_G8,),
            in_specs=[
                pl.BlockSpec((2, _RB, din), lambda i: (0, i, 0)),
                pl.BlockSpec((_RB, dout), lambda i: (i, 0)),
                pl.BlockSpec((din, dout), lambda i: (0, 0)),
                pl.BlockSpec((1, dout), lambda i: (0, 0)),
            ],
            out_specs=[
                pl.BlockSpec((_RB, dout), lambda i: (i, 0)),
                pl.BlockSpec((1, dout), lambda i: (0, 0)),
                pl.BlockSpec((1, dout), lambda i: (0, 0)),
            ],
            out_shape=[
                jax.ShapeDtypeStruct((_N, dout), jnp.float32),
                jax.ShapeDtypeStruct((1, dout), jnp.float32),
                jax.ShapeDtypeStruct((1, dout), jnp.float32),
            ],
        )(parts, root, wrel, brel)

    return call


_NEDGE_TOT = 15360
_NWORK = 32
_EPW = _NEDGE_TOT // _NWORK
_ECH = 96
_NCH = _EPW // _ECH
_RPW = _N // 16


def _mk_gconv_sc(d):
    mesh = plsc.VectorSubcoreMesh(core_axis_name="c", subcore_axis_name="s")

    @functools.partial(
        pl.kernel,
        out_type=jax.ShapeDtypeStruct((2, _N, d), jnp.float32),
        mesh=mesh,
        scratch_types=[
            pltpu.VMEM((_EPW,), jnp.int32),
            pltpu.VMEM((_NCH, _ECH), jnp.int32),
            pltpu.VMEM((_EPW,), jnp.float32),
            pltpu.VMEM((_EPW, d), jnp.float32),
            pltpu.VMEM_SHARED((_N, d), jnp.float32),
            pltpu.SemaphoreType.DMA,
            pltpu.SemaphoreType.DMA,
        ],
    )
    def k(h_hbm, src_hbm, dst_hbm, w_hbm, z_hbm, out_hbm,
          sidx, didx, wv, rows, agg, sem, zsem):
        c = lax.axis_index("c")
        s = lax.axis_index("s")
        wid = s * 2 + c
        base = wid * _EPW
        zcp = pltpu.async_copy(z_hbm.at[pl.ds(s * _RPW, _RPW)],
                               agg.at[pl.ds(s * _RPW, _RPW)], zsem)
        pltpu.sync_copy(src_hbm.at[pl.ds(base, _EPW)], sidx)
        for j in range(_NCH):
            pltpu.sync_copy(dst_hbm.at[pl.ds(base + _ECH * j, _ECH)], didx.at[j])
        pltpu.sync_copy(w_hbm.at[pl.ds(base, _EPW)], wv)
        cps = [pltpu.async_copy(h_hbm.at[sidx.at[pl.ds(_ECH * j, _ECH)]],
                                rows.at[pl.ds(_ECH * j, _ECH)], sem)
               for j in range(_NCH)]
        for cp in cps:
            cp.wait()

        def scale(g, carry):
            wg = wv[pl.ds(g * 16, 16)]
            for i in range(16):
                e = g * 16 + i
                wvec = jnp.full((16,), 0.0, jnp.float32) + wg[i]
                for q in range(d // 16):
                    sl = pl.ds(q * 16, 16)
                    rows[e, sl] = rows[e, sl] * wvec
            return carry

        lax.fori_loop(0, _EPW // 16, scale, 0)
        zcp.wait()
        plsc.subcore_barrier()
        scps = [pltpu.async_copy(rows.at[pl.ds(_ECH * j, _ECH)],
                                 agg.at[didx.at[j]], sem, add=True)
                for j in range(_NCH)]
        for cp in scps:
            cp.wait()
        plsc.subcore_barrier()
        pltpu.sync_copy(agg.at[pl.ds(s * _RPW, _RPW)],
                        out_hbm.at[c].at[pl.ds(s * _RPW, _RPW)])

    return k


def _pool_body(h_ref, s_ref, q_ref, g_ref, be_ref, out_ref):
    m = s_ref[...] / float(_N)
    v = q_ref[...] / float(_N) - m * m
    sc = lax.rsqrt(v + 1e-5) * g_ref[...]
    sh = be_ref[...] - m * sc
    hn = h_ref[...] * sc + sh
    out_ref[...] = jnp.max(hn.reshape(_RB // _NEL, _NEL, 64), axis=1)


def _maxpool(h, s, q, g, be):
    gspec = pl.BlockSpec((1, 64), lambda i: (0, 0))
    return pl.pallas_call(
        _pool_body,
        grid=(_G8,),
        in_specs=[pl.BlockSpec((_RB, 64), lambda i: (i, 0)),
                  gspec, gspec, gspec, gspec],
        out_specs=pl.BlockSpec((_RB // _NEL, 64), lambda i: (i, 0)),
        out_shape=jax.ShapeDtypeStruct((_B, 64), jnp.float32),
    )(h, s, q, g.reshape(1, 64), be.reshape(1, 64))


def _heads_body(p_ref, w5_ref, b5_ref, wc_ref, bc_ref, out_ref):
    feat = jnp.dot(p_ref[...], w5_ref[...], preferred_element_type=jnp.float32)
    feat = jnp.maximum(feat + b5_ref[...], 0.0)
    z = jnp.dot(feat, wc_ref[...], preferred_element_type=jnp.float32) + bc_ref[...]
    ps = []
    for k in range(3):
        zp = z[:, 2 * k:2 * k + 2]
        m = jnp.max(zp, axis=1, keepdims=True)
        e = jnp.exp(zp - m)
        ps.append(e / jnp.sum(e, axis=1, keepdims=True))
    p0, p1, p2 = ps
    p_hc = p0[:, 0:1] * p1[:, 0:1]
    p_ad = p0[:, 1:2] * p2[:, 1:2]
    p_ftd = p0[:, 0:1] * p1[:, 1:2] + p0[:, 1:2] * p2[:, 0:1]
    out_ref[...] = jnp.log(jnp.concatenate([p_hc, p_ftd, p_ad], axis=1) + 1e-8)


def _heads(pooled, w5, b5, wcat, bcat):
    return pl.pallas_call(
        _heads_body,
        out_shape=jax.ShapeDtypeStruct((_B, 3), jnp.float32),
    )(pooled, w5, b5, wcat, bcat)


def kernel(x, edge_index, batch, W2, b2, g3, be3, W3, b3, g4, be4, W4, b4,
           g5, be5, ew1, Wrel1, brel1, Wroot1, g6, be6, ew2, Wrel2, brel2,
           Wroot2, g7, be7, W5, b5, Whr, bhr, Whf, bhf, Wfa, bfa):
    w2pad = jnp.concatenate(
        [W2, jnp.zeros(((_NCHUNK + 1) * 128 - _NPOOL, 512), jnp.float32)], axis=0)
    h1, rs1, rq1 = _stage1(jnp.swapaxes(x, 0, 1), w2pad, b2.reshape(1, 512))
    h2, rs2, rq2, w1t, w2t = _mk_mlp_bn(512, 256, _B * 512.0, wexp=True)(
        h1, rs1, rq1, g3, be3, W3, b3.reshape(1, 256),
        ew1.reshape(1, 60), ew2.reshape(1, 60))
    h3, rs3, rq3 = _mk_mlp_bn(256, 128, _B * 256.0)(
        h2, rs2, rq2, g4, be4, W4, b4.reshape(1, 128))
    h3n, root1 = _mk_apply_root(128, 64, True)(h3, rs3, rq3, g5, be5, Wroot1)

    src = edge_index[0]
    dst = edge_index[1]
    w1e = w1t.reshape(_NEDGE_TOT)
    w2e = w2t.reshape(_NEDGE_TOT)

    z128 = jnp.zeros((_N, 128), jnp.float32)
    parts1 = _mk_gconv_sc(128)(h3n, src, dst, w1e, z128)
    h4, s4, q4 = _mk_combine(128, 64)(parts1, root1, Wrel1, brel1.reshape(1, 64))
    h4n, root2 = _mk_apply_root(64, 64, False, dpad=128)(
        h4, s4, q4, g6.reshape(1, 64), be6.reshape(1, 64), Wroot2)

    parts2 = _mk_gconv_sc(128)(h4n, src, dst, w2e, z128)
    wrel2p = jnp.concatenate([Wrel2, jnp.zeros((64, 64), jnp.float32)], axis=0)
    h5, s5, q5 = _mk_combine(128, 64)(parts2, root2, wrel2p, brel2.reshape(1, 64))
    pooled = _maxpool(h5, s5, q5, g7, be7)
    wcat = jnp.concatenate([Whr, Whf, Wfa], axis=1)
    bcat = jnp.concatenate([bhr, bhf, bfa]).reshape(1, 6)
    return _heads(pooled, W5, b5.reshape(1, 32), wcat, bcat)


# speedup vs baseline: 3.6632x; 1.0287x over previous
import functools

import jax
import jax.numpy as jnp
from jax import lax
from jax.experimental import pallas as pl
from jax.experimental.pallas import tpu as pltpu
from jax.experimental.pallas import tpu_sc as plsc

_B = 256
_NEL = 19
_N = _B * _NEL
_D0 = 20000
_WLEN = 25
_NPOOL = 800
_CHUNK = 3200
_NCHUNK = 6
_RB = 608
_G8 = _N // _RB


def _pool_mats():
    j = jnp.arange(_CHUNK)
    pc = (j[:, None] // _WLEN == jnp.arange(128)[None, :]).astype(jnp.float32) / _WLEN
    jt = jnp.arange(_D0 - _NCHUNK * _CHUNK)
    pt = (jt[:, None] // _WLEN == jnp.arange(128)[None, :]).astype(jnp.float32) / _WLEN
    return pc, pt


def _onehot_el():
    return (jnp.arange(_N)[:, None] % _NEL == jnp.arange(_NEL)[None, :]
            ).astype(jnp.float32)


_KW = 256


def _k1a_body(xt_ref, pc_ref, pt_ref, w2_ref, b2_ref,
              h1_ref, rs_ref, rq_ref, pooled_ref):
    dn = (((0,), (0,)), ((), ()))
    for c in range(_NCHUNK):
        pooled_ref[:, 128 * c:128 * (c + 1)] = lax.dot_general(
            xt_ref[pl.ds(_CHUNK * c, _CHUNK), :], pc_ref[...], dn,
            preferred_element_type=jnp.float32)
    pooled_ref[:, _NCHUNK * 128:(_NCHUNK + 1) * 128] = lax.dot_general(
        xt_ref[pl.ds(_NCHUNK * _CHUNK, _D0 - _NCHUNK * _CHUNK), :], pt_ref[...],
        dn, preferred_element_type=jnp.float32)
    h = jnp.dot(pooled_ref[...], w2_ref[...], preferred_element_type=jnp.float32)
    h = jnp.maximum(h + b2_ref[...], 0.0)
    h1_ref[...] = h
    rs_ref[...] = jnp.sum(h, axis=1, keepdims=True)
    rq_ref[...] = jnp.sum(h * h, axis=1, keepdims=True)


def _stage1(xt, w2pad, b2):
    pc, pt = _pool_mats()
    nblk = _N // _KW
    return pl.pallas_call(
        _k1a_body,
        grid=(nblk,),
        in_specs=[
            pl.BlockSpec((_D0, _KW), lambda i: (0, i)),
            pl.BlockSpec((_CHUNK, 128), lambda i: (0, 0)),
            pl.BlockSpec((_D0 - _NCHUNK * _CHUNK, 128), lambda i: (0, 0)),
            pl.BlockSpec(((_NCHUNK + 1) * 128, 512), lambda i: (0, 0)),
            pl.BlockSpec((1, 512), lambda i: (0, 0)),
        ],
        out_specs=[
            pl.BlockSpec((_KW, 512), lambda i: (i, 0)),
            pl.BlockSpec((_KW, 1), lambda i: (i, 0)),
            pl.BlockSpec((_KW, 1), lambda i: (i, 0)),
        ],
        out_shape=[
            jax.ShapeDtypeStruct((_N, 512), jnp.float32),
            jax.ShapeDtypeStruct((_N, 1), jnp.float32),
            jax.ShapeDtypeStruct((_N, 1), jnp.float32),
        ],
        scratch_shapes=[pltpu.VMEM((_KW, (_NCHUNK + 1) * 128), jnp.float32)],
    )(xt, pc, pt, w2pad, b2)


def _resolve_cols(rs_ref, rq_ref, oh_ref, g_ref, be_ref, count, scc_ref, shc_ref):
    oh = oh_ref[...]
    dn0 = (((0,), (0,)), ((), ()))
    s19 = lax.dot_general(rs_ref[...], oh, dn0, preferred_element_type=jnp.float32)
    q19 = lax.dot_general(rq_ref[...], oh, dn0, preferred_element_type=jnp.float32)
    m = s19 / count
    v = q19 / count - m * m
    inv = lax.rsqrt(v + 1e-5) * g_ref[...]
    sh = be_ref[...] - m * inv
    dn1 = (((1,), (1,)), ((), ()))
    scc_ref[...] = lax.dot_general(oh, inv, dn1, preferred_element_type=jnp.float32)
    shc_ref[...] = lax.dot_general(oh, sh, dn1, preferred_element_type=jnp.float32)


def _mk_mlp_bn(din, dout, count, wexp=False):
    def body(*refs):
        if wexp:
            (h_ref, rs_ref, rq_ref, oh_ref, g_ref, be_ref, w_ref, b_ref,
             ew1_ref, ew2_ref, out_ref, rso_ref, rqo_ref, w1t_ref, w2t_ref,
             scc_ref, shc_ref) = refs
        else:
            (h_ref, rs_ref, rq_ref, oh_ref, g_ref, be_ref, w_ref, b_ref,
             out_ref, rso_ref, rqo_ref, scc_ref, shc_ref) = refs
        i = pl.program_id(0)

        @pl.when(i == 0)
        def _resolve():
            _resolve_cols(rs_ref, rq_ref, oh_ref, g_ref, be_ref, count,
                          scc_ref, shc_ref)
            if wexp:
                for ew_ref, wt_ref in ((ew1_ref, w1t_ref), (ew2_ref, w2t_ref)):
                    ew = ew_ref[...]
                    sp = jnp.maximum(ew, 0.0) + jnp.log1p(jnp.exp(-jnp.abs(ew)))
                    wt_ref[...] = jnp.broadcast_to(sp, (_B, 60))

        off = pl.multiple_of(i * _RB, _RB)
        sc = scc_ref[pl.ds(off, _RB), :]
        sh = shc_ref[pl.ds(off, _RB), :]
        hb = h_ref[...] * sc + sh
        h2 = jnp.dot(hb, w_ref[...], preferred_element_type=jnp.float32)
        h2 = jnp.maximum(h2 + b_ref[...], 0.0)
        out_ref[...] = h2
        rso_ref[...] = jnp.sum(h2, axis=1, keepdims=True)
        rqo_ref[...] = jnp.sum(h2 * h2, axis=1, keepdims=True)

    full = lambda shape: pl.BlockSpec(shape, lambda i: tuple(0 for _ in shape))
    out_shape = [
        jax.ShapeDtypeStruct((_N, dout), jnp.float32),
        jax.ShapeDtypeStruct((_N, 1), jnp.float32),
        jax.ShapeDtypeStruct((_N, 1), jnp.float32),
    ]
    out_specs = [
        pl.BlockSpec((_RB, dout), lambda i: (i, 0)),
        pl.BlockSpec((_RB, 1), lambda i: (i, 0)),
        pl.BlockSpec((_RB, 1), lambda i: (i, 0)),
    ]
    if wexp:
        out_shape += [jax.ShapeDtypeStruct((_B, 60), jnp.float32)] * 2
        out_specs += [full((_B, 60))] * 2

    def call(h, rs, rq, g, be, w, b, *ews):
        return pl.pallas_call(
            body,
            grid=(_G8,),
            in_specs=[
                pl.BlockSpec((_RB, din), lambda i: (i, 0)),
                full((_N, 1)), full((_N, 1)), full((_N, _NEL)),
                full((1, _NEL)), full((1, _NEL)),
                full((din, dout)), full((1, dout)),
            ] + ([full((1, 60))] * 2 if wexp else []),
            out_specs=out_specs,
            out_shape=out_shape,
            scratch_shapes=[pltpu.VMEM((_N, 1), jnp.float32),
                            pltpu.VMEM((_N, 1), jnp.float32)],
        )(h, rs, rq, _onehot_el(), g.reshape(1, _NEL), be.reshape(1, _NEL),
          w, b, *ews)

    return call


def _mk_apply_root(din, dmid, col_scale, dpad=None):
    dpad = dpad or din

    def body(h_ref, a_ref, b_ref, oh_ref, g_ref, be_ref, wr_ref,
             hn_ref, root_ref, scc_ref, shc_ref):
        i = pl.program_id(0)
        if col_scale:
            @pl.when(i == 0)
            def _resolve():
                _resolve_cols(a_ref, b_ref, oh_ref, g_ref, be_ref,
                              _B * float(din), scc_ref, shc_ref)

            off = pl.multiple_of(i * _RB, _RB)
            sc = scc_ref[pl.ds(off, _RB), :]
            sh = shc_ref[pl.ds(off, _RB), :]
        else:
            m = a_ref[...] / float(_N)
            v = b_ref[...] / float(_N) - m * m
            sc = lax.rsqrt(v + 1e-5) * g_ref[...]
            sh = be_ref[...] - m * sc
        hn = h_ref[...] * sc + sh
        if dpad > din:
            hn_ref[:, :din] = hn
            hn_ref[:, din:] = jnp.zeros((_RB, dpad - din), jnp.float32)
        else:
            hn_ref[...] = hn
        root_ref[...] = jnp.dot(hn, wr_ref[...], preferred_element_type=jnp.float32)

    full = lambda shape: pl.BlockSpec(shape, lambda i: tuple(0 for _ in shape))
    sspec = (full((_N, 1)) if col_scale else full((1, din)))
    gspec = (full((1, _NEL)) if col_scale else full((1, din)))

    def call(h, a, b, g, be, wroot):
        return pl.pallas_call(
            body,
            grid=(_G8,),
            in_specs=[
                pl.BlockSpec((_RB, din), lambda i: (i, 0)),
                sspec, sspec, full((_N, _NEL)), gspec, gspec,
                full((din, dmid)),
            ],
            out_specs=[
                pl.BlockSpec((_RB, dpad), lambda i: (i, 0)),
                pl.BlockSpec((_RB, dmid), lambda i: (i, 0)),
            ],
            out_shape=[
                jax.ShapeDtypeStruct((_N, dpad), jnp.float32),
                jax.ShapeDtypeStruct((_N, dmid), jnp.float32),
            ],
            scratch_shapes=[pltpu.VMEM((_N, 1), jnp.float32),
                            pltpu.VMEM((_N, 1), jnp.float32)],
        )(h, a, b, _onehot_el(),
          g.reshape(1, -1) if g.ndim == 1 else g,
          be.reshape(1, -1) if be.ndim == 1 else be, wroot)

    return call


def _mk_combine(din, dout, pool_out=False):
    def body(p_ref, root_ref, wrel_ref, brel_ref, out_ref, *rest):
        if pool_out:
            mn_ref, s_ref, q_ref = rest
        else:
            s_ref, q_ref = rest
        agg = p_ref[0] + p_ref[1]
        h = jnp.dot(agg, wrel_ref[...], preferred_element_type=jnp.float32)
        h = jnp.maximum(h + brel_ref[...] + root_ref[...], 0.0)
        if pool_out:
            h3 = h.reshape(_RB // _NEL, _NEL, dout)
            out_ref[...] = jnp.max(h3, axis=1)
            mn_ref[...] = jnp.min(h3, axis=1)
        else:
            out_ref[...] = h
        ps = jnp.sum(h, axis=0).reshape(1, dout)
        pq = jnp.sum(h * h, axis=0).reshape(1, dout)

        @pl.when(pl.program_id(0) == 0)
        def _init():
            s_ref[...] = jnp.zeros_like(s_ref)
            q_ref[...] = jnp.zeros_like(q_ref)

        s_ref[...] += ps
        q_ref[...] += pq

    hspec = (pl.BlockSpec((_RB // _NEL, dout), lambda i: (i, 0)) if pool_out
             else pl.BlockSpec((_RB, dout), lambda i: (i, 0)))
    hshape = ((_B, dout) if pool_out else (_N, dout))
    out_specs = [hspec] + ([hspec] if pool_out else []) + [
        pl.BlockSpec((1, dout), lambda i: (0, 0)),
        pl.BlockSpec((1, dout), lambda i: (0, 0)),
    ]
    out_shape = ([jax.ShapeDtypeStruct(hshape, jnp.float32)] *
                 (2 if pool_out else 1)) + [
        jax.ShapeDtypeStruct((1, dout), jnp.float32),
        jax.ShapeDtypeStruct((1, dout), jnp.float32),
    ]

    def call(parts, root, wrel, brel):
        return pl.pallas_call(
            body,
            grid=(_G8,),
            in_specs=[
                pl.BlockSpec((2, _RB, din), lambda i: (0, i, 0)),
                pl.BlockSpec((_RB, dout), lambda i: (i, 0)),
                pl.BlockSpec((din, dout), lambda i: (0, 0)),
                pl.BlockSpec((1, dout), lambda i: (0, 0)),
            ],
            out_specs=out_specs,
            out_shape=out_shape,
        )(parts, root, wrel, brel)

    return call


_NEDGE_TOT = 15360
_NWORK = 32
_EPW = _NEDGE_TOT // _NWORK
_ECH = 96
_NCH = _EPW // _ECH
_RPW = _N // 16


def _mk_gconv_sc(d):
    mesh = plsc.VectorSubcoreMesh(core_axis_name="c", subcore_axis_name="s")

    @functools.partial(
        pl.kernel,
        out_type=jax.ShapeDtypeStruct((2, _N, d), jnp.float32),
        mesh=mesh,
        scratch_types=[
            pltpu.VMEM((_EPW,), jnp.int32),
            pltpu.VMEM((_NCH, _ECH), jnp.int32),
            pltpu.VMEM((_EPW,), jnp.float32),
            pltpu.VMEM((_EPW, d), jnp.float32),
            pltpu.VMEM_SHARED((_N, d), jnp.float32),
            pltpu.SemaphoreType.DMA,
            pltpu.SemaphoreType.DMA,
        ],
    )
    def k(h_hbm, src_hbm, dst_hbm, w_hbm, z_hbm, out_hbm,
          sidx, didx, wv, rows, agg, sem, zsem):
        c = lax.axis_index("c")
        s = lax.axis_index("s")
        wid = s * 2 + c
        base = wid * _EPW
        zcp = pltpu.async_copy(z_hbm.at[pl.ds(s * _RPW, _RPW)],
                               agg.at[pl.ds(s * _RPW, _RPW)], zsem)
        pltpu.sync_copy(src_hbm.at[pl.ds(base, _EPW)], sidx)
        for j in range(_NCH):
            pltpu.sync_copy(dst_hbm.at[pl.ds(base + _ECH * j, _ECH)], didx.at[j])
        pltpu.sync_copy(w_hbm.at[pl.ds(base, _EPW)], wv)
        cps = [pltpu.async_copy(h_hbm.at[sidx.at[pl.ds(_ECH * j, _ECH)]],
                                rows.at[pl.ds(_ECH * j, _ECH)], sem)
               for j in range(_NCH)]

        gpc = _ECH // 16

        def scale(g, carry):
            wg = wv[pl.ds(g * 16, 16)]
            for i in range(16):
                e = g * 16 + i
                wvec = jnp.full((16,), 0.0, jnp.float32) + wg[i]
                for q in range(d // 16):
                    sl = pl.ds(q * 16, 16)
                    rows[e, sl] = rows[e, sl] * wvec
            return carry

        for j in range(_NCH):
            cps[j].wait()
            lax.fori_loop(j * gpc, (j + 1) * gpc, scale, 0)
        zcp.wait()
        plsc.subcore_barrier()
        scps = [pltpu.async_copy(rows.at[pl.ds(_ECH * j, _ECH)],
                                 agg.at[didx.at[j]], sem, add=True)
                for j in range(_NCH)]
        for cp in scps:
            cp.wait()
        plsc.subcore_barrier()
        pltpu.sync_copy(agg.at[pl.ds(s * _RPW, _RPW)],
                        out_hbm.at[c].at[pl.ds(s * _RPW, _RPW)])

    return k


def _heads_body(mx_ref, mn_ref, s_ref, q_ref, g_ref, be_ref,
                w5_ref, b5_ref, wc_ref, bc_ref, out_ref):
    m = s_ref[...] / float(_N)
    v = q_ref[...] / float(_N) - m * m
    sc = lax.rsqrt(v + 1e-5) * g_ref[...]
    sh = be_ref[...] - m * sc
    pooled = jnp.where(sc >= 0.0, mx_ref[...] * sc, mn_ref[...] * sc) + sh
    feat = jnp.dot(pooled, w5_ref[...], preferred_element_type=jnp.float32)
    feat = jnp.maximum(feat + b5_ref[...], 0.0)
    z = jnp.dot(feat, wc_ref[...], preferred_element_type=jnp.float32) + bc_ref[...]
    ps = []
    for k in range(3):
        zp = z[:, 2 * k:2 * k + 2]
        m = jnp.max(zp, axis=1, keepdims=True)
        e = jnp.exp(zp - m)
        ps.append(e / jnp.sum(e, axis=1, keepdims=True))
    p0, p1, p2 = ps
    p_hc = p0[:, 0:1] * p1[:, 0:1]
    p_ad = p0[:, 1:2] * p2[:, 1:2]
    p_ftd = p0[:, 0:1] * p1[:, 1:2] + p0[:, 1:2] * p2[:, 0:1]
    out_ref[...] = jnp.log(jnp.concatenate([p_hc, p_ftd, p_ad], axis=1) + 1e-8)


def _heads(mx, mn, s5, q5, g7, be7, w5, b5, wcat, bcat):
    return pl.pallas_call(
        _heads_body,
        out_shape=jax.ShapeDtypeStruct((_B, 3), jnp.float32),
    )(mx, mn, s5, q5, g7.reshape(1, 64), be7.reshape(1, 64), w5, b5, wcat, bcat)


def kernel(x, edge_index, batch, W2, b2, g3, be3, W3, b3, g4, be4, W4, b4,
           g5, be5, ew1, Wrel1, brel1, Wroot1, g6, be6, ew2, Wrel2, brel2,
           Wroot2, g7, be7, W5, b5, Whr, bhr, Whf, bhf, Wfa, bfa):
    w2pad = jnp.concatenate(
        [W2, jnp.zeros(((_NCHUNK + 1) * 128 - _NPOOL, 512), jnp.float32)], axis=0)
    h1, rs1, rq1 = _stage1(jnp.swapaxes(x, 0, 1), w2pad, b2.reshape(1, 512))
    h2, rs2, rq2, w1t, w2t = _mk_mlp_bn(512, 256, _B * 512.0, wexp=True)(
        h1, rs1, rq1, g3, be3, W3, b3.reshape(1, 256),
        ew1.reshape(1, 60), ew2.reshape(1, 60))
    h3, rs3, rq3 = _mk_mlp_bn(256, 128, _B * 256.0)(
        h2, rs2, rq2, g4, be4, W4, b4.reshape(1, 128))
    h3n, root1 = _mk_apply_root(128, 64, True)(h3, rs3, rq3, g5, be5, Wroot1)

    src = edge_index[0]
    dst = edge_index[1]
    w1e = w1t.reshape(_NEDGE_TOT)
    w2e = w2t.reshape(_NEDGE_TOT)

    z128 = jnp.zeros((_N, 128), jnp.float32)
    parts1 = _mk_gconv_sc(128)(h3n, src, dst, w1e, z128)
    h4, s4, q4 = _mk_combine(128, 64)(parts1, root1, Wrel1, brel1.reshape(1, 64))
    h4n, root2 = _mk_apply_root(64, 64, False, dpad=128)(
        h4, s4, q4, g6.reshape(1, 64), be6.reshape(1, 64), Wroot2)

    parts2 = _mk_gconv_sc(128)(h4n, src, dst, w2e, z128)
    wrel2p = jnp.concatenate([Wrel2, jnp.zeros((64, 64), jnp.float32)], axis=0)
    mx, mn, s5, q5 = _mk_combine(128, 64, pool_out=True)(
        parts2, root2, wrel2p, brel2.reshape(1, 64))
    wcat = jnp.concatenate([Whr, Whf, Wfa], axis=1)
    bcat = jnp.concatenate([bhr, bhf, bfa]).reshape(1, 6)
    return _heads(mx, mn, s5, q5, g7, be7, W5, b5.reshape(1, 32), wcat, bcat)


# speedup vs baseline: 3.8532x; 1.0519x over previous
import functools

import jax
import jax.numpy as jnp
from jax import lax
from jax.experimental import pallas as pl
from jax.experimental.pallas import tpu as pltpu
from jax.experimental.pallas import tpu_sc as plsc

_B = 256
_NEL = 19
_N = _B * _NEL
_D0 = 20000
_WLEN = 25
_NPOOL = 800
_CHUNK = 3200
_NCHUNK = 6
_RB = 1216
_G8 = _N // _RB


def _pool_mats():
    j = jnp.arange(_CHUNK)
    pc = (j[:, None] // _WLEN == jnp.arange(128)[None, :]).astype(jnp.float32) / _WLEN
    jt = jnp.arange(_D0 - _NCHUNK * _CHUNK)
    pt = (jt[:, None] // _WLEN == jnp.arange(128)[None, :]).astype(jnp.float32) / _WLEN
    return pc, pt


def _onehot_el():
    return (jnp.arange(_N)[:, None] % _NEL == jnp.arange(_NEL)[None, :]
            ).astype(jnp.float32)


_KW = 256


def _k1a_body(xt_ref, pc_ref, pt_ref, w2_ref, b2_ref,
              h1_ref, rs_ref, rq_ref, pooled_ref):
    dn = (((0,), (0,)), ((), ()))
    for c in range(_NCHUNK):
        pooled_ref[:, 128 * c:128 * (c + 1)] = lax.dot_general(
            xt_ref[pl.ds(_CHUNK * c, _CHUNK), :], pc_ref[...], dn,
            preferred_element_type=jnp.float32)
    pooled_ref[:, _NCHUNK * 128:(_NCHUNK + 1) * 128] = lax.dot_general(
        xt_ref[pl.ds(_NCHUNK * _CHUNK, _D0 - _NCHUNK * _CHUNK), :], pt_ref[...],
        dn, preferred_element_type=jnp.float32)
    h = jnp.dot(pooled_ref[...], w2_ref[...], preferred_element_type=jnp.float32)
    h = jnp.maximum(h + b2_ref[...], 0.0)
    h1_ref[...] = h
    rs_ref[...] = jnp.sum(h, axis=1, keepdims=True)
    rq_ref[...] = jnp.sum(h * h, axis=1, keepdims=True)


def _stage1(xt, w2pad, b2):
    pc, pt = _pool_mats()
    nblk = _N // _KW
    return pl.pallas_call(
        _k1a_body,
        grid=(nblk,),
        in_specs=[
            pl.BlockSpec((_D0, _KW), lambda i: (0, i)),
            pl.BlockSpec((_CHUNK, 128), lambda i: (0, 0)),
            pl.BlockSpec((_D0 - _NCHUNK * _CHUNK, 128), lambda i: (0, 0)),
            pl.BlockSpec(((_NCHUNK + 1) * 128, 512), lambda i: (0, 0)),
            pl.BlockSpec((1, 512), lambda i: (0, 0)),
        ],
        out_specs=[
            pl.BlockSpec((_KW, 512), lambda i: (i, 0)),
            pl.BlockSpec((_KW, 1), lambda i: (i, 0)),
            pl.BlockSpec((_KW, 1), lambda i: (i, 0)),
        ],
        out_shape=[
            jax.ShapeDtypeStruct((_N, 512), jnp.float32),
            jax.ShapeDtypeStruct((_N, 1), jnp.float32),
            jax.ShapeDtypeStruct((_N, 1), jnp.float32),
        ],
        scratch_shapes=[pltpu.VMEM((_KW, (_NCHUNK + 1) * 128), jnp.float32)],
    )(xt, pc, pt, w2pad, b2)


def _resolve_cols(rs_ref, rq_ref, oh_ref, g_ref, be_ref, count, scc_ref, shc_ref):
    oh = oh_ref[...]
    dn0 = (((0,), (0,)), ((), ()))
    s19 = lax.dot_general(rs_ref[...], oh, dn0, preferred_element_type=jnp.float32)
    q19 = lax.dot_general(rq_ref[...], oh, dn0, preferred_element_type=jnp.float32)
    m = s19 / count
    v = q19 / count - m * m
    inv = lax.rsqrt(v + 1e-5) * g_ref[...]
    sh = be_ref[...] - m * inv
    dn1 = (((1,), (1,)), ((), ()))
    scc_ref[...] = lax.dot_general(oh, inv, dn1, preferred_element_type=jnp.float32)
    shc_ref[...] = lax.dot_general(oh, sh, dn1, preferred_element_type=jnp.float32)


def _mk_mlp_bn(din, dout, count, wexp=False):
    def body(*refs):
        if wexp:
            (h_ref, rs_ref, rq_ref, oh_ref, g_ref, be_ref, w_ref, b_ref,
             ew1_ref, ew2_ref, out_ref, rso_ref, rqo_ref, w1t_ref, w2t_ref,
             scc_ref, shc_ref) = refs
        else:
            (h_ref, rs_ref, rq_ref, oh_ref, g_ref, be_ref, w_ref, b_ref,
             out_ref, rso_ref, rqo_ref, scc_ref, shc_ref) = refs
        i = pl.program_id(0)

        @pl.when(i == 0)
        def _resolve():
            _resolve_cols(rs_ref, rq_ref, oh_ref, g_ref, be_ref, count,
                          scc_ref, shc_ref)
            if wexp:
                for ew_ref, wt_ref in ((ew1_ref, w1t_ref), (ew2_ref, w2t_ref)):
                    ew = ew_ref[...]
                    sp = jnp.maximum(ew, 0.0) + jnp.log1p(jnp.exp(-jnp.abs(ew)))
                    wt_ref[...] = jnp.broadcast_to(sp, (_B, 60))

        off = pl.multiple_of(i * _RB, _RB)
        sc = scc_ref[pl.ds(off, _RB), :]
        sh = shc_ref[pl.ds(off, _RB), :]
        hb = h_ref[...] * sc + sh
        h2 = jnp.dot(hb, w_ref[...], preferred_element_type=jnp.float32)
        h2 = jnp.maximum(h2 + b_ref[...], 0.0)
        out_ref[...] = h2
        rso_ref[...] = jnp.sum(h2, axis=1, keepdims=True)
        rqo_ref[...] = jnp.sum(h2 * h2, axis=1, keepdims=True)

    full = lambda shape: pl.BlockSpec(shape, lambda i: tuple(0 for _ in shape))
    out_shape = [
        jax.ShapeDtypeStruct((_N, dout), jnp.float32),
        jax.ShapeDtypeStruct((_N, 1), jnp.float32),
        jax.ShapeDtypeStruct((_N, 1), jnp.float32),
    ]
    out_specs = [
        pl.BlockSpec((_RB, dout), lambda i: (i, 0)),
        pl.BlockSpec((_RB, 1), lambda i: (i, 0)),
        pl.BlockSpec((_RB, 1), lambda i: (i, 0)),
    ]
    if wexp:
        out_shape += [jax.ShapeDtypeStruct((_B, 60), jnp.float32)] * 2
        out_specs += [full((_B, 60))] * 2

    def call(h, rs, rq, g, be, w, b, *ews):
        return pl.pallas_call(
            body,
            grid=(_G8,),
            in_specs=[
                pl.BlockSpec((_RB, din), lambda i: (i, 0)),
                full((_N, 1)), full((_N, 1)), full((_N, _NEL)),
                full((1, _NEL)), full((1, _NEL)),
                full((din, dout)), full((1, dout)),
            ] + ([full((1, 60))] * 2 if wexp else []),
            out_specs=out_specs,
            out_shape=out_shape,
            scratch_shapes=[pltpu.VMEM((_N, 1), jnp.float32),
                            pltpu.VMEM((_N, 1), jnp.float32)],
        )(h, rs, rq, _onehot_el(), g.reshape(1, _NEL), be.reshape(1, _NEL),
          w, b, *ews)

    return call


def _mk_apply_root(din, dmid, col_scale, dpad=None):
    dpad = dpad or din

    def body(h_ref, a_ref, b_ref, oh_ref, g_ref, be_ref, wr_ref,
             hn_ref, root_ref, scc_ref, shc_ref):
        i = pl.program_id(0)
        if col_scale:
            @pl.when(i == 0)
            def _resolve():
                _resolve_cols(a_ref, b_ref, oh_ref, g_ref, be_ref,
                              _B * float(din), scc_ref, shc_ref)

            off = pl.multiple_of(i * _RB, _RB)
            sc = scc_ref[pl.ds(off, _RB), :]
            sh = shc_ref[pl.ds(off, _RB), :]
        else:
            m = a_ref[...] / float(_N)
            v = b_ref[...] / float(_N) - m * m
            sc = lax.rsqrt(v + 1e-5) * g_ref[...]
            sh = be_ref[...] - m * sc
        hn = h_ref[...] * sc + sh
        if dpad > din:
            hn_ref[:, :din] = hn
            hn_ref[:, din:] = jnp.zeros((_RB, dpad - din), jnp.float32)
        else:
            hn_ref[...] = hn
        root_ref[...] = jnp.dot(hn, wr_ref[...], preferred_element_type=jnp.float32)

    full = lambda shape: pl.BlockSpec(shape, lambda i: tuple(0 for _ in shape))
    sspec = (full((_N, 1)) if col_scale else full((1, din)))
    gspec = (full((1, _NEL)) if col_scale else full((1, din)))

    def call(h, a, b, g, be, wroot):
        return pl.pallas_call(
            body,
            grid=(_G8,),
            in_specs=[
                pl.BlockSpec((_RB, din), lambda i: (i, 0)),
                sspec, sspec, full((_N, _NEL)), gspec, gspec,
                full((din, dmid)),
            ],
            out_specs=[
                pl.BlockSpec((_RB, dpad), lambda i: (i, 0)),
                pl.BlockSpec((_RB, dmid), lambda i: (i, 0)),
            ],
            out_shape=[
                jax.ShapeDtypeStruct((_N, dpad), jnp.float32),
                jax.ShapeDtypeStruct((_N, dmid), jnp.float32),
            ],
            scratch_shapes=[pltpu.VMEM((_N, 1), jnp.float32),
                            pltpu.VMEM((_N, 1), jnp.float32)],
        )(h, a, b, _onehot_el(),
          g.reshape(1, -1) if g.ndim == 1 else g,
          be.reshape(1, -1) if be.ndim == 1 else be, wroot)

    return call


def _mk_combine(din, dout, pool_out=False):
    def body(p_ref, root_ref, wrel_ref, brel_ref, out_ref, *rest):
        if pool_out:
            mn_ref, s_ref, q_ref = rest
        else:
            s_ref, q_ref = rest
        agg = p_ref[0] + p_ref[1]
        h = jnp.dot(agg, wrel_ref[...], preferred_element_type=jnp.float32)
        h = jnp.maximum(h + brel_ref[...] + root_ref[...], 0.0)
        if pool_out:
            h3 = h.reshape(_RB // _NEL, _NEL, dout)
            out_ref[...] = jnp.max(h3, axis=1)
            mn_ref[...] = jnp.min(h3, axis=1)
        else:
            out_ref[...] = h
        ps = jnp.sum(h, axis=0).reshape(1, dout)
        pq = jnp.sum(h * h, axis=0).reshape(1, dout)

        @pl.when(pl.program_id(0) == 0)
        def _init():
            s_ref[...] = jnp.zeros_like(s_ref)
            q_ref[...] = jnp.zeros_like(q_ref)

        s_ref[...] += ps
        q_ref[...] += pq

    hspec = (pl.BlockSpec((_RB // _NEL, dout), lambda i: (i, 0)) if pool_out
             else pl.BlockSpec((_RB, dout), lambda i: (i, 0)))
    hshape = ((_B, dout) if pool_out else (_N, dout))
    out_specs = [hspec] + ([hspec] if pool_out else []) + [
        pl.BlockSpec((1, dout), lambda i: (0, 0)),
        pl.BlockSpec((1, dout), lambda i: (0, 0)),
    ]
    out_shape = ([jax.ShapeDtypeStruct(hshape, jnp.float32)] *
                 (2 if pool_out else 1)) + [
        jax.ShapeDtypeStruct((1, dout), jnp.float32),
        jax.ShapeDtypeStruct((1, dout), jnp.float32),
    ]

    def call(parts, root, wrel, brel):
        return pl.pallas_call(
            body,
            grid=(_G8,),
            in_specs=[
                pl.BlockSpec((2, _RB, din), lambda i: (0, i, 0)),
                pl.BlockSpec((_RB, dout), lambda i: (i, 0)),
                pl.BlockSpec((din, dout), lambda i: (0, 0)),
                pl.BlockSpec((1, dout), lambda i: (0, 0)),
            ],
            out_specs=out_specs,
            out_shape=out_shape,
        )(parts, root, wrel, brel)

    return call


_NEDGE_TOT = 15360
_NWORK = 32
_EPW = _NEDGE_TOT // _NWORK
_ECH = 96
_NCH = _EPW // _ECH
_RPW = _N // 16


def _mk_gconv_sc(d):
    mesh = plsc.VectorSubcoreMesh(core_axis_name="c", subcore_axis_name="s")

    @functools.partial(
        pl.kernel,
        out_type=jax.ShapeDtypeStruct((2, _N, d), jnp.float32),
        mesh=mesh,
        scratch_types=[
            pltpu.VMEM((_EPW,), jnp.int32),
            pltpu.VMEM((_NCH, _ECH), jnp.int32),
            pltpu.VMEM((_EPW,), jnp.float32),
            pltpu.VMEM((_EPW, d), jnp.float32),
            pltpu.VMEM_SHARED((_N, d), jnp.float32),
            pltpu.SemaphoreType.DMA,
            pltpu.SemaphoreType.DMA,
        ],
    )
    def k(h_hbm, src_hbm, dst_hbm, w_hbm, z_hbm, out_hbm,
          sidx, didx, wv, rows, agg, sem, zsem):
        c = lax.axis_index("c")
        s = lax.axis_index("s")
        wid = s * 2 + c
        base = wid * _EPW
        zcp = pltpu.async_copy(z_hbm.at[pl.ds(s * _RPW, _RPW)],
                               agg.at[pl.ds(s * _RPW, _RPW)], zsem)
        pltpu.sync_copy(src_hbm.at[pl.ds(base, _EPW)], sidx)
        for j in range(_NCH):
            pltpu.sync_copy(dst_hbm.at[pl.ds(base + _ECH * j, _ECH)], didx.at[j])
        pltpu.sync_copy(w_hbm.at[pl.ds(base, _EPW)], wv)
        cps = [pltpu.async_copy(h_hbm.at[sidx.at[pl.ds(_ECH * j, _ECH)]],
                                rows.at[pl.ds(_ECH * j, _ECH)], sem)
               for j in range(_NCH)]

        gpc = _ECH // 16

        def scale(g, carry):
            wg = wv[pl.ds(g * 16, 16)]
            for i in range(16):
                e = g * 16 + i
                wvec = jnp.full((16,), 0.0, jnp.float32) + wg[i]
                for q in range(d // 16):
                    sl = pl.ds(q * 16, 16)
                    rows[e, sl] = rows[e, sl] * wvec
            return carry

        for j in range(_NCH):
            cps[j].wait()
            lax.fori_loop(j * gpc, (j + 1) * gpc, scale, 0)
        zcp.wait()
        plsc.subcore_barrier()
        scps = [pltpu.async_copy(rows.at[pl.ds(_ECH * j, _ECH)],
                                 agg.at[didx.at[j]], sem, add=True)
                for j in range(_NCH)]
        for cp in scps:
            cp.wait()
        plsc.subcore_barrier()
        pltpu.sync_copy(agg.at[pl.ds(s * _RPW, _RPW)],
                        out_hbm.at[c].at[pl.ds(s * _RPW, _RPW)])

    return k


def _heads_body(mx_ref, mn_ref, s_ref, q_ref, g_ref, be_ref,
                w5_ref, b5_ref, wc_ref, bc_ref, out_ref):
    m = s_ref[...] / float(_N)
    v = q_ref[...] / float(_N) - m * m
    sc = lax.rsqrt(v + 1e-5) * g_ref[...]
    sh = be_ref[...] - m * sc
    pooled = jnp.where(sc >= 0.0, mx_ref[...] * sc, mn_ref[...] * sc) + sh
    feat = jnp.dot(pooled, w5_ref[...], preferred_element_type=jnp.float32)
    feat = jnp.maximum(feat + b5_ref[...], 0.0)
    z = jnp.dot(feat, wc_ref[...], preferred_element_type=jnp.float32) + bc_ref[...]
    ps = []
    for k in range(3):
        zp = z[:, 2 * k:2 * k + 2]
        m = jnp.max(zp, axis=1, keepdims=True)
        e = jnp.exp(zp - m)
        ps.append(e / jnp.sum(e, axis=1, keepdims=True))
    p0, p1, p2 = ps
    p_hc = p0[:, 0:1] * p1[:, 0:1]
    p_ad = p0[:, 1:2] * p2[:, 1:2]
    p_ftd = p0[:, 0:1] * p1[:, 1:2] + p0[:, 1:2] * p2[:, 0:1]
    out_ref[...] = jnp.log(jnp.concatenate([p_hc, p_ftd, p_ad], axis=1) + 1e-8)


def _heads(mx, mn, s5, q5, g7, be7, w5, b5, wcat, bcat):
    return pl.pallas_call(
        _heads_body,
        out_shape=jax.ShapeDtypeStruct((_B, 3), jnp.float32),
    )(mx, mn, s5, q5, g7.reshape(1, 64), be7.reshape(1, 64), w5, b5, wcat, bcat)


def kernel(x, edge_index, batch, W2, b2, g3, be3, W3, b3, g4, be4, W4, b4,
           g5, be5, ew1, Wrel1, brel1, Wroot1, g6, be6, ew2, Wrel2, brel2,
           Wroot2, g7, be7, W5, b5, Whr, bhr, Whf, bhf, Wfa, bfa):
    w2pad = jnp.concatenate(
        [W2, jnp.zeros(((_NCHUNK + 1) * 128 - _NPOOL, 512), jnp.float32)], axis=0)
    h1, rs1, rq1 = _stage1(jnp.swapaxes(x, 0, 1), w2pad, b2.reshape(1, 512))
    h2, rs2, rq2, w1t, w2t = _mk_mlp_bn(512, 256, _B * 512.0, wexp=True)(
        h1, rs1, rq1, g3, be3, W3, b3.reshape(1, 256),
        ew1.reshape(1, 60), ew2.reshape(1, 60))
    h3, rs3, rq3 = _mk_mlp_bn(256, 128, _B * 256.0)(
        h2, rs2, rq2, g4, be4, W4, b4.reshape(1, 128))
    h3n, root1 = _mk_apply_root(128, 64, True)(h3, rs3, rq3, g5, be5, Wroot1)

    src = edge_index[0]
    dst = edge_index[1]
    w1e = w1t.reshape(_NEDGE_TOT)
    w2e = w2t.reshape(_NEDGE_TOT)

    z128 = jnp.zeros((_N, 128), jnp.float32)
    parts1 = _mk_gconv_sc(128)(h3n, src, dst, w1e, z128)
    h4, s4, q4 = _mk_combine(128, 64)(parts1, root1, Wrel1, brel1.reshape(1, 64))
    h4n, root2 = _mk_apply_root(64, 64, False, dpad=128)(
        h4, s4, q4, g6.reshape(1, 64), be6.reshape(1, 64), Wroot2)

    parts2 = _mk_gconv_sc(128)(h4n, src, dst, w2e, z128)
    wrel2p = jnp.concatenate([Wrel2, jnp.zeros((64, 64), jnp.float32)], axis=0)
    mx, mn, s5, q5 = _mk_combine(128, 64, pool_out=True)(
        parts2, root2, wrel2p, brel2.reshape(1, 64))
    wcat = jnp.concatenate([Whr, Whf, Wfa], axis=1)
    bcat = jnp.concatenate([bhr, bhf, bfa]).reshape(1, 6)
    return _heads(mx, mn, s5, q5, g7, be7, W5, b5.reshape(1, 32), wcat, bcat)


# speedup vs baseline: 3.9688x; 1.0300x over previous
import functools

import jax
import jax.numpy as jnp
from jax import lax
from jax.experimental import pallas as pl
from jax.experimental.pallas import tpu as pltpu
from jax.experimental.pallas import tpu_sc as plsc

_B = 256
_NEL = 19
_N = _B * _NEL
_D0 = 20000
_WLEN = 25
_NPOOL = 800
_CHUNK = 3200
_NCHUNK = 6
_RB = 2432
_G8 = _N // _RB


def _pool_mats():
    j = jnp.arange(_CHUNK)
    pc = (j[:, None] // _WLEN == jnp.arange(128)[None, :]).astype(jnp.float32) / _WLEN
    jt = jnp.arange(_D0 - _NCHUNK * _CHUNK)
    pt = (jt[:, None] // _WLEN == jnp.arange(128)[None, :]).astype(jnp.float32) / _WLEN
    return pc, pt


def _onehot_el():
    return (jnp.arange(_N)[:, None] % _NEL == jnp.arange(_NEL)[None, :]
            ).astype(jnp.float32)


_KW = 256


def _k1a_body(xt_ref, pc_ref, pt_ref, w2_ref, b2_ref,
              h1_ref, rs_ref, rq_ref, pooled_ref):
    dn = (((0,), (0,)), ((), ()))
    for c in range(_NCHUNK):
        pooled_ref[:, 128 * c:128 * (c + 1)] = lax.dot_general(
            xt_ref[pl.ds(_CHUNK * c, _CHUNK), :], pc_ref[...], dn,
            preferred_element_type=jnp.float32)
    pooled_ref[:, _NCHUNK * 128:(_NCHUNK + 1) * 128] = lax.dot_general(
        xt_ref[pl.ds(_NCHUNK * _CHUNK, _D0 - _NCHUNK * _CHUNK), :], pt_ref[...],
        dn, preferred_element_type=jnp.float32)
    h = jnp.dot(pooled_ref[...], w2_ref[...], preferred_element_type=jnp.float32)
    h = jnp.maximum(h + b2_ref[...], 0.0)
    h1_ref[...] = h
    rs_ref[...] = jnp.sum(h, axis=1, keepdims=True)
    rq_ref[...] = jnp.sum(h * h, axis=1, keepdims=True)


def _stage1(xt, w2pad, b2):
    pc, pt = _pool_mats()
    nblk = _N // _KW
    return pl.pallas_call(
        _k1a_body,
        grid=(nblk,),
        in_specs=[
            pl.BlockSpec((_D0, _KW), lambda i: (0, i)),
            pl.BlockSpec((_CHUNK, 128), lambda i: (0, 0)),
            pl.BlockSpec((_D0 - _NCHUNK * _CHUNK, 128), lambda i: (0, 0)),
            pl.BlockSpec(((_NCHUNK + 1) * 128, 512), lambda i: (0, 0)),
            pl.BlockSpec((1, 512), lambda i: (0, 0)),
        ],
        out_specs=[
            pl.BlockSpec((_KW, 512), lambda i: (i, 0)),
            pl.BlockSpec((_KW, 1), lambda i: (i, 0)),
            pl.BlockSpec((_KW, 1), lambda i: (i, 0)),
        ],
        out_shape=[
            jax.ShapeDtypeStruct((_N, 512), jnp.float32),
            jax.ShapeDtypeStruct((_N, 1), jnp.float32),
            jax.ShapeDtypeStruct((_N, 1), jnp.float32),
        ],
        scratch_shapes=[pltpu.VMEM((_KW, (_NCHUNK + 1) * 128), jnp.float32)],
    )(xt, pc, pt, w2pad, b2)


def _resolve_cols(rs_ref, rq_ref, oh_ref, g_ref, be_ref, count, scc_ref, shc_ref):
    oh = oh_ref[...]
    dn0 = (((0,), (0,)), ((), ()))
    s19 = lax.dot_general(rs_ref[...], oh, dn0, preferred_element_type=jnp.float32)
    q19 = lax.dot_general(rq_ref[...], oh, dn0, preferred_element_type=jnp.float32)
    m = s19 / count
    v = q19 / count - m * m
    inv = lax.rsqrt(v + 1e-5) * g_ref[...]
    sh = be_ref[...] - m * inv
    dn1 = (((1,), (1,)), ((), ()))
    scc_ref[...] = lax.dot_general(oh, inv, dn1, preferred_element_type=jnp.float32)
    shc_ref[...] = lax.dot_general(oh, sh, dn1, preferred_element_type=jnp.float32)


def _mk_mlp_bn(din, dout, count, wexp=False):
    def body(*refs):
        if wexp:
            (h_ref, rs_ref, rq_ref, oh_ref, g_ref, be_ref, w_ref, b_ref,
             ew1_ref, ew2_ref, out_ref, rso_ref, rqo_ref, w1t_ref, w2t_ref,
             scc_ref, shc_ref) = refs
        else:
            (h_ref, rs_ref, rq_ref, oh_ref, g_ref, be_ref, w_ref, b_ref,
             out_ref, rso_ref, rqo_ref, scc_ref, shc_ref) = refs
        i = pl.program_id(0)

        @pl.when(i == 0)
        def _resolve():
            _resolve_cols(rs_ref, rq_ref, oh_ref, g_ref, be_ref, count,
                          scc_ref, shc_ref)
            if wexp:
                for ew_ref, wt_ref in ((ew1_ref, w1t_ref), (ew2_ref, w2t_ref)):
                    ew = ew_ref[...]
                    sp = jnp.maximum(ew, 0.0) + jnp.log1p(jnp.exp(-jnp.abs(ew)))
                    wt_ref[...] = jnp.broadcast_to(sp, (_B, 60))

        off = pl.multiple_of(i * _RB, _RB)
        sc = scc_ref[pl.ds(off, _RB), :]
        sh = shc_ref[pl.ds(off, _RB), :]
        hb = h_ref[...] * sc + sh
        h2 = jnp.dot(hb, w_ref[...], preferred_element_type=jnp.float32)
        h2 = jnp.maximum(h2 + b_ref[...], 0.0)
        out_ref[...] = h2
        rso_ref[...] = jnp.sum(h2, axis=1, keepdims=True)
        rqo_ref[...] = jnp.sum(h2 * h2, axis=1, keepdims=True)

    full = lambda shape: pl.BlockSpec(shape, lambda i: tuple(0 for _ in shape))
    out_shape = [
        jax.ShapeDtypeStruct((_N, dout), jnp.float32),
        jax.ShapeDtypeStruct((_N, 1), jnp.float32),
        jax.ShapeDtypeStruct((_N, 1), jnp.float32),
    ]
    out_specs = [
        pl.BlockSpec((_RB, dout), lambda i: (i, 0)),
        pl.BlockSpec((_RB, 1), lambda i: (i, 0)),
        pl.BlockSpec((_RB, 1), lambda i: (i, 0)),
    ]
    if wexp:
        out_shape += [jax.ShapeDtypeStruct((_B, 60), jnp.float32)] * 2
        out_specs += [full((_B, 60))] * 2

    def call(h, rs, rq, g, be, w, b, *ews):
        return pl.pallas_call(
            body,
            grid=(_G8,),
            in_specs=[
                pl.BlockSpec((_RB, din), lambda i: (i, 0)),
                full((_N, 1)), full((_N, 1)), full((_N, _NEL)),
                full((1, _NEL)), full((1, _NEL)),
                full((din, dout)), full((1, dout)),
            ] + ([full((1, 60))] * 2 if wexp else []),
            out_specs=out_specs,
            out_shape=out_shape,
            scratch_shapes=[pltpu.VMEM((_N, 1), jnp.float32),
                            pltpu.VMEM((_N, 1), jnp.float32)],
        )(h, rs, rq, _onehot_el(), g.reshape(1, _NEL), be.reshape(1, _NEL),
          w, b, *ews)

    return call


def _mk_apply_root(din, dmid, col_scale, dpad=None):
    dpad = dpad or din

    def body(h_ref, a_ref, b_ref, oh_ref, g_ref, be_ref, wr_ref,
             hn_ref, root_ref, scc_ref, shc_ref):
        i = pl.program_id(0)
        if col_scale:
            @pl.when(i == 0)
            def _resolve():
                _resolve_cols(a_ref, b_ref, oh_ref, g_ref, be_ref,
                              _B * float(din), scc_ref, shc_ref)

            off = pl.multiple_of(i * _RB, _RB)
            sc = scc_ref[pl.ds(off, _RB), :]
            sh = shc_ref[pl.ds(off, _RB), :]
        else:
            m = a_ref[...] / float(_N)
            v = b_ref[...] / float(_N) - m * m
            sc = lax.rsqrt(v + 1e-5) * g_ref[...]
            sh = be_ref[...] - m * sc
        hn = h_ref[...] * sc + sh
        if dpad > din:
            hn_ref[:, :din] = hn
            hn_ref[:, din:] = jnp.zeros((_RB, dpad - din), jnp.float32)
        else:
            hn_ref[...] = hn
        root_ref[...] = jnp.dot(hn, wr_ref[...], preferred_element_type=jnp.float32)

    full = lambda shape: pl.BlockSpec(shape, lambda i: tuple(0 for _ in shape))
    sspec = (full((_N, 1)) if col_scale else full((1, din)))
    gspec = (full((1, _NEL)) if col_scale else full((1, din)))

    def call(h, a, b, g, be, wroot):
        return pl.pallas_call(
            body,
            grid=(_G8,),
            in_specs=[
                pl.BlockSpec((_RB, din), lambda i: (i, 0)),
                sspec, sspec, full((_N, _NEL)), gspec, gspec,
                full((din, dmid)),
            ],
            out_specs=[
                pl.BlockSpec((_RB, dpad), lambda i: (i, 0)),
                pl.BlockSpec((_RB, dmid), lambda i: (i, 0)),
            ],
            out_shape=[
                jax.ShapeDtypeStruct((_N, dpad), jnp.float32),
                jax.ShapeDtypeStruct((_N, dmid), jnp.float32),
            ],
            scratch_shapes=[pltpu.VMEM((_N, 1), jnp.float32),
                            pltpu.VMEM((_N, 1), jnp.float32)],
        )(h, a, b, _onehot_el(),
          g.reshape(1, -1) if g.ndim == 1 else g,
          be.reshape(1, -1) if be.ndim == 1 else be, wroot)

    return call


def _mk_combine(din, dout, pool_out=False):
    def body(p_ref, root_ref, wrel_ref, brel_ref, out_ref, *rest):
        if pool_out:
            mn_ref, s_ref, q_ref = rest
        else:
            s_ref, q_ref = rest
        agg = p_ref[0] + p_ref[1]
        h = jnp.dot(agg, wrel_ref[...], preferred_element_type=jnp.float32)
        h = jnp.maximum(h + brel_ref[...] + root_ref[...], 0.0)
        if pool_out:
            h3 = h.reshape(_RB // _NEL, _NEL, dout)
            out_ref[...] = jnp.max(h3, axis=1)
            mn_ref[...] = jnp.min(h3, axis=1)
        else:
            out_ref[...] = h
        ps = jnp.sum(h, axis=0).reshape(1, dout)
        pq = jnp.sum(h * h, axis=0).reshape(1, dout)

        @pl.when(pl.program_id(0) == 0)
        def _init():
            s_ref[...] = jnp.zeros_like(s_ref)
            q_ref[...] = jnp.zeros_like(q_ref)

        s_ref[...] += ps
        q_ref[...] += pq

    hspec = (pl.BlockSpec((_RB // _NEL, dout), lambda i: (i, 0)) if pool_out
             else pl.BlockSpec((_RB, dout), lambda i: (i, 0)))
    hshape = ((_B, dout) if pool_out else (_N, dout))
    out_specs = [hspec] + ([hspec] if pool_out else []) + [
        pl.BlockSpec((1, dout), lambda i: (0, 0)),
        pl.BlockSpec((1, dout), lambda i: (0, 0)),
    ]
    out_shape = ([jax.ShapeDtypeStruct(hshape, jnp.float32)] *
                 (2 if pool_out else 1)) + [
        jax.ShapeDtypeStruct((1, dout), jnp.float32),
        jax.ShapeDtypeStruct((1, dout), jnp.float32),
    ]

    def call(parts, root, wrel, brel):
        return pl.pallas_call(
            body,
            grid=(_G8,),
            in_specs=[
                pl.BlockSpec((2, _RB, din), lambda i: (0, i, 0)),
                pl.BlockSpec((_RB, dout), lambda i: (i, 0)),
                pl.BlockSpec((din, dout), lambda i: (0, 0)),
                pl.BlockSpec((1, dout), lambda i: (0, 0)),
            ],
            out_specs=out_specs,
            out_shape=out_shape,
        )(parts, root, wrel, brel)

    return call


_NEDGE_TOT = 15360
_NWORK = 32
_EPW = _NEDGE_TOT // _NWORK
_ECH = 96
_NCH = _EPW // _ECH
_RPW = _N // 16


def _mk_gconv_sc(d):
    mesh = plsc.VectorSubcoreMesh(core_axis_name="c", subcore_axis_name="s")

    @functools.partial(
        pl.kernel,
        out_type=jax.ShapeDtypeStruct((2, _N, d), jnp.float32),
        mesh=mesh,
        scratch_types=[
            pltpu.VMEM((_EPW,), jnp.int32),
            pltpu.VMEM((_NCH, _ECH), jnp.int32),
            pltpu.VMEM((_EPW,), jnp.float32),
            pltpu.VMEM((_EPW, d), jnp.float32),
            pltpu.VMEM_SHARED((_N, d), jnp.float32),
            pltpu.SemaphoreType.DMA,
            pltpu.SemaphoreType.DMA,
        ],
    )
    def k(h_hbm, src_hbm, dst_hbm, w_hbm, z_hbm, out_hbm,
          sidx, didx, wv, rows, agg, sem, zsem):
        c = lax.axis_index("c")
        s = lax.axis_index("s")
        wid = s * 2 + c
        base = wid * _EPW
        zcp = pltpu.async_copy(z_hbm.at[pl.ds(s * _RPW, _RPW)],
                               agg.at[pl.ds(s * _RPW, _RPW)], zsem)
        pltpu.sync_copy(src_hbm.at[pl.ds(base, _EPW)], sidx)
        for j in range(_NCH):
            pltpu.sync_copy(dst_hbm.at[pl.ds(base + _ECH * j, _ECH)], didx.at[j])
        pltpu.sync_copy(w_hbm.at[pl.ds(base, _EPW)], wv)
        cps = [pltpu.async_copy(h_hbm.at[sidx.at[pl.ds(_ECH * j, _ECH)]],
                                rows.at[pl.ds(_ECH * j, _ECH)], sem)
               for j in range(_NCH)]

        gpc = _ECH // 16

        def scale(g, carry):
            wg = wv[pl.ds(g * 16, 16)]
            for i in range(16):
                e = g * 16 + i
                wvec = jnp.full((16,), 0.0, jnp.float32) + wg[i]
                for q in range(d // 16):
                    sl = pl.ds(q * 16, 16)
                    rows[e, sl] = rows[e, sl] * wvec
            return carry

        for j in range(_NCH):
            cps[j].wait()
            lax.fori_loop(j * gpc, (j + 1) * gpc, scale, 0)
        zcp.wait()
        plsc.subcore_barrier()
        scps = [pltpu.async_copy(rows.at[pl.ds(_ECH * j, _ECH)],
                                 agg.at[didx.at[j]], sem, add=True)
                for j in range(_NCH)]
        for cp in scps:
            cp.wait()
        plsc.subcore_barrier()
        pltpu.sync_copy(agg.at[pl.ds(s * _RPW, _RPW)],
                        out_hbm.at[c].at[pl.ds(s * _RPW, _RPW)])

    return k


def _heads_body(mx_ref, mn_ref, s_ref, q_ref, g_ref, be_ref,
                w5_ref, b5_ref, wc_ref, bc_ref, out_ref):
    m = s_ref[...] / float(_N)
    v = q_ref[...] / float(_N) - m * m
    sc = lax.rsqrt(v + 1e-5) * g_ref[...]
    sh = be_ref[...] - m * sc
    pooled = jnp.where(sc >= 0.0, mx_ref[...] * sc, mn_ref[...] * sc) + sh
    feat = jnp.dot(pooled, w5_ref[...], preferred_element_type=jnp.float32)
    feat = jnp.maximum(feat + b5_ref[...], 0.0)
    z = jnp.dot(feat, wc_ref[...], preferred_element_type=jnp.float32) + bc_ref[...]
    ps = []
    for k in range(3):
        zp = z[:, 2 * k:2 * k + 2]
        m = jnp.max(zp, axis=1, keepdims=True)
        e = jnp.exp(zp - m)
        ps.append(e / jnp.sum(e, axis=1, keepdims=True))
    p0, p1, p2 = ps
    p_hc = p0[:, 0:1] * p1[:, 0:1]
    p_ad = p0[:, 1:2] * p2[:, 1:2]
    p_ftd = p0[:, 0:1] * p1[:, 1:2] + p0[:, 1:2] * p2[:, 0:1]
    out_ref[...] = jnp.log(jnp.concatenate([p_hc, p_ftd, p_ad], axis=1) + 1e-8)


def _heads(mx, mn, s5, q5, g7, be7, w5, b5, wcat, bcat):
    return pl.pallas_call(
        _heads_body,
        out_shape=jax.ShapeDtypeStruct((_B, 3), jnp.float32),
    )(mx, mn, s5, q5, g7.reshape(1, 64), be7.reshape(1, 64), w5, b5, wcat, bcat)


def kernel(x, edge_index, batch, W2, b2, g3, be3, W3, b3, g4, be4, W4, b4,
           g5, be5, ew1, Wrel1, brel1, Wroot1, g6, be6, ew2, Wrel2, brel2,
           Wroot2, g7, be7, W5, b5, Whr, bhr, Whf, bhf, Wfa, bfa):
    w2pad = jnp.concatenate(
        [W2, jnp.zeros(((_NCHUNK + 1) * 128 - _NPOOL, 512), jnp.float32)], axis=0)
    h1, rs1, rq1 = _stage1(jnp.swapaxes(x, 0, 1), w2pad, b2.reshape(1, 512))
    h2, rs2, rq2, w1t, w2t = _mk_mlp_bn(512, 256, _B * 512.0, wexp=True)(
        h1, rs1, rq1, g3, be3, W3, b3.reshape(1, 256),
        ew1.reshape(1, 60), ew2.reshape(1, 60))
    h3, rs3, rq3 = _mk_mlp_bn(256, 128, _B * 256.0)(
        h2, rs2, rq2, g4, be4, W4, b4.reshape(1, 128))
    h3n, root1 = _mk_apply_root(128, 64, True)(h3, rs3, rq3, g5, be5, Wroot1)

    src = edge_index[0]
    dst = edge_index[1]
    w1e = w1t.reshape(_NEDGE_TOT)
    w2e = w2t.reshape(_NEDGE_TOT)

    z128 = jnp.zeros((_N, 128), jnp.float32)
    parts1 = _mk_gconv_sc(128)(h3n, src, dst, w1e, z128)
    h4, s4, q4 = _mk_combine(128, 64)(parts1, root1, Wrel1, brel1.reshape(1, 64))
    h4n, root2 = _mk_apply_root(64, 64, False, dpad=128)(
        h4, s4, q4, g6.reshape(1, 64), be6.reshape(1, 64), Wroot2)

    parts2 = _mk_gconv_sc(128)(h4n, src, dst, w2e, z128)
    wrel2p = jnp.concatenate([Wrel2, jnp.zeros((64, 64), jnp.float32)], axis=0)
    mx, mn, s5, q5 = _mk_combine(128, 64, pool_out=True)(
        parts2, root2, wrel2p, brel2.reshape(1, 64))
    wcat = jnp.concatenate([Whr, Whf, Wfa], axis=1)
    bcat = jnp.concatenate([bhr, bhf, bfa]).reshape(1, 6)
    return _heads(mx, mn, s5, q5, g7, be7, W5, b5.reshape(1, 32), wcat, bcat)
